# Initial kernel scaffold; baseline (speedup 1.0000x reference)
import jax
import jax.numpy as jnp
from jax.experimental import pallas as pl


def kernel(x_p, edge_index_p, x_l, edge_index_l, bfs_init, Wp1, bp1, Wp2, bp2, Wl1, bl1, Wl2, bl2, Wd1, bd1, Wd2, bd2, Wf, bf):
    raise NotImplementedError("write your pallas kernel here")


# speedup vs baseline: 3.0342x; 3.0342x over previous
import functools

import jax
import jax.numpy as jnp
from jax.experimental import pallas as pl
from jax.experimental.pallas import tpu as pltpu

_NP = 10000
_NL = 10000
_ND = 10001
_NATOM = 11


def _mm_kernel(x_ref, w_ref, o_ref):
    o_ref[...] = jnp.dot(x_ref[...], w_ref[...],
                         preferred_element_type=jnp.float32)


def _pallas_matmul(x, w):
    m, k = x.shape
    _, n = w.shape
    bm = 1024
    mp = ((m + bm - 1) // bm) * bm
    xp = jnp.pad(x, ((0, mp - m), (0, 0)))
    out = pl.pallas_call(
        _mm_kernel,
        grid=(mp // bm,),
        in_specs=[pl.BlockSpec((bm, k), lambda i: (i, 0)),
                  pl.BlockSpec((k, n), lambda i: (0, 0))],
        out_specs=pl.BlockSpec((bm, n), lambda i: (i, 0)),
        out_shape=jax.ShapeDtypeStruct((mp, n), jnp.float32),
    )(xp, w)
    return out[:m]


def _gcn_spmv(y, src, dst, n):
    return jnp.zeros((n, y.shape[1]), y.dtype).at[dst].add(y[src]) + y


def kernel(x_p, edge_index_p, x_l, edge_index_l, bfs_init, Wp1, bp1, Wp2,
           bp2, Wl1, bl1, Wl2, bl2, Wd1, bd1, Wd2, bd2, Wf, bf):
    src_p, dst_p = edge_index_p[0], edge_index_p[1]
    src_l, dst_l = edge_index_l[0], edge_index_l[1]

    deg_p = jnp.zeros((_NP,), jnp.float32).at[dst_p].add(1.0) + 1.0
    inv_p = jax.lax.rsqrt(deg_p)
    y_p = x_p * inv_p[:, None]
    agg1 = inv_p[:, None] * _gcn_spmv(y_p, src_p, dst_p, _NP)
    h_p = jax.nn.relu(_pallas_matmul(agg1, Wp1) + bp1)
    g_p = jnp.zeros((_NP,), jnp.float32).at[src_p].add(inv_p[dst_p])
    c_p = inv_p * (g_p + inv_p)
    z_pocket = (c_p @ h_p / _NP) @ Wp2 + bp2

    deg_l = jnp.zeros((_NL,), jnp.float32).at[dst_l].add(1.0) + 1.0
    inv_l = jax.lax.rsqrt(deg_l)
    y_l = x_l * inv_l[:, None]
    aggl1 = inv_l[:, None] * _gcn_spmv(y_l, src_l, dst_l, _NL)
    h_l = jax.nn.relu(_pallas_matmul(aggl1, Wl1) + bl1)
    y2 = h_l * inv_l[:, None]
    aggl2 = inv_l[:, None] * _gcn_spmv(y2, src_l, dst_l, _NL)
    z_lig = _pallas_matmul(aggl2, Wl2) + bl2

    logits = z_lig @ Wf + bf
    logits = logits.at[:, -1].add(-1e9)
    m = jnp.max(logits, axis=1, keepdims=True)
    e = jnp.exp(logits - m)
    p = e / jnp.sum(e, axis=1, keepdims=True)
    lab_v = x_l[:, 4:]
    log_prob = jnp.sum(jnp.log(jnp.sum(p * lab_v, axis=1)))
    H_init = jnp.concatenate([jnp.mean(z_lig, axis=0),
                              jnp.mean(lab_v, axis=0)])

    src0 = bfs_init[0, 0]
    dst0 = bfs_init[1, 0]
    l_stop = jnp.zeros((_NATOM,), jnp.float32).at[_NATOM - 1].set(1.0)
    lab_v2 = jnp.concatenate([lab_v, l_stop[None, :]], axis=0)
    isq = 1.0 / jnp.sqrt(2.0)
    same = src0 == dst0
    x_s0 = lab_v2[src0]
    x_d0 = lab_v2[dst0]
    agg_d = jnp.where(same, x_d0, x_s0 * isq + x_d0 * 0.5)
    h_full = jax.nn.relu(_pallas_matmul(lab_v2, Wd1) + bd1)
    h_spec = jax.nn.relu(agg_d @ Wd1 + bd1)
    h = h_full.at[dst0].set(h_spec)
    z_v = _pallas_matmul(h, Wd2) + bd2
    agg2_d = jnp.where(same, h_spec, h[src0] * isq + h_spec * 0.5)
    z_v = z_v.at[dst0].set(agg2_d @ Wd2 + bd2)
    H_t = jnp.concatenate([jnp.mean(z_v, axis=0), jnp.mean(lab_v2, axis=0)])

    return (log_prob, z_pocket, z_v, H_init, H_t)


# speedup vs baseline: 22.4783x; 7.4082x over previous
import functools

import jax
import jax.numpy as jnp
from jax import lax
from jax.experimental import pallas as pl
from jax.experimental.pallas import tpu as pltpu
from jax.experimental.pallas import tpu_sc as plsc

_N = 10000
_E = 320000
_NPAD = 10240
_NATOM = 11
_ND = 10001

_NC, _NS, _LANES = 2, 16, 16
_NW = _NC * _NS
_EPT = _E // _NW
_CH = 80
_NCHUNK = _EPT // _CH
_RPT = _NPAD // _NS

_BM = 512
_GRID = _NPAD // _BM

_f32 = jnp.float32

_sc_mesh = plsc.VectorSubcoreMesh(
    core_axis_name="c", subcore_axis_name="s",
    num_cores=_NC, num_subcores=_NS)
_sc_params = pltpu.CompilerParams(use_tc_tiling_on_sc=False)


def _fill_1d(buf, n, value):
    v = jnp.full((_LANES,), value, _f32)

    def body(i, _):
        buf[pl.ds(i * _LANES, _LANES)] = v
        return 0

    lax.fori_loop(0, n // _LANES, body, 0)


def _fill_2d(buf, rows, cols, value):
    v = jnp.full((_LANES,), value, _f32)
    nseg = cols // _LANES

    def body(i, _):
        for k in range(nseg):
            buf[i, pl.ds(k * _LANES, _LANES)] = v
        return 0

    lax.fori_loop(0, rows, body, 0)


def _deg_body(dstp_hbm, dstl_hbm, outp_hbm, outl_hbm,
              idx_v, ones_v, zed_v, histp_s, histl_s):
    c = lax.axis_index("c")
    s = lax.axis_index("s")
    _fill_1d(zed_v, _RPT, 0.0)
    _fill_1d(ones_v, _CH, 1.0)
    pltpu.sync_copy(zed_v, histp_s.at[pl.ds(s * _RPT, _RPT)])
    pltpu.sync_copy(zed_v, histl_s.at[pl.ds(s * _RPT, _RPT)])
    plsc.subcore_barrier()

    def scatter_ones(hist_s):
        def body(j, _):
            pltpu.sync_copy(ones_v, hist_s.at[idx_v.at[j]], add=True)
            return 0
        lax.fori_loop(0, _NCHUNK, body, 0)

    pltpu.sync_copy(dstp_hbm.at[c, s], idx_v)
    scatter_ones(histp_s)
    pltpu.sync_copy(dstl_hbm.at[c, s], idx_v)
    scatter_ones(histl_s)
    plsc.subcore_barrier()
    sl = pl.ds(s * _RPT, _RPT)
    pltpu.sync_copy(histp_s.at[sl], outp_hbm.at[c, sl])
    pltpu.sync_copy(histl_s.at[sl], outl_hbm.at[c, sl])


_deg_call = pl.kernel(
    _deg_body,
    out_type=[jax.ShapeDtypeStruct((_NC, _NPAD), _f32),
              jax.ShapeDtypeStruct((_NC, _NPAD), _f32)],
    mesh=_sc_mesh,
    compiler_params=_sc_params,
    scratch_types=[
        pltpu.VMEM((_NCHUNK, _CH), jnp.int32),
        pltpu.VMEM((_CH,), _f32),
        pltpu.VMEM((_RPT,), _f32),
        pltpu.VMEM_SHARED((_NPAD,), _f32),
        pltpu.VMEM_SHARED((_NPAD,), _f32),
    ])


def _spmv_loop(y_hbm, src_hbm, dst_hbm, c, s, sidx, didx, gbuf, acc_s, sem,
               g=None):
    pltpu.sync_copy(src_hbm.at[c, s], sidx)
    pltpu.sync_copy(dst_hbm.at[c, s], didx)

    def body(j, _):
        pltpu.async_copy(y_hbm.at[sidx.at[j]], gbuf, sem).wait()
        pltpu.sync_copy(gbuf, acc_s.at[didx.at[j]], add=True)
        if g is not None:
            inv_hbm, gvals, gacc_s = g
            pltpu.async_copy(inv_hbm.at[didx.at[j]], gvals, sem).wait()
            pltpu.sync_copy(gvals, gacc_s.at[sidx.at[j]], add=True)
        return 0

    lax.fori_loop(0, _NCHUNK, body, 0)


def _mid_body(yp_hbm, srcp_hbm, dstp_hbm, inv_hbm, yl_hbm, srcl_hbm,
              dstl_hbm, aggp_hbm, g_hbm, aggl_hbm,
              sidx, didx, gbuf, gbuf16, gvals,
              accp_s, gacc_s, accl_s, sem):
    c = lax.axis_index("c")
    s = lax.axis_index("s")
    _fill_2d(gbuf, _CH, 128, 0.0)
    _fill_2d(gbuf16, _CH, 16, 0.0)
    _fill_1d(gvals, _CH, 0.0)
    for k in range(_RPT // _CH):
        pltpu.sync_copy(gbuf, accp_s.at[pl.ds(s * _RPT + k * _CH, _CH)])
        pltpu.sync_copy(gbuf16, accl_s.at[pl.ds(s * _RPT + k * _CH, _CH)])
        pltpu.sync_copy(gvals, gacc_s.at[pl.ds(s * _RPT + k * _CH, _CH)])
    plsc.subcore_barrier()

    _spmv_loop(yp_hbm, srcp_hbm, dstp_hbm, c, s, sidx, didx, gbuf, accp_s,
               sem, g=(inv_hbm, gvals, gacc_s))
    _spmv_loop(yl_hbm, srcl_hbm, dstl_hbm, c, s, sidx, didx, gbuf16, accl_s,
               sem)
    plsc.subcore_barrier()
    sl = pl.ds(s * _RPT, _RPT)
    pltpu.sync_copy(accp_s.at[sl], aggp_hbm.at[c, sl])
    pltpu.sync_copy(gacc_s.at[sl], g_hbm.at[c, sl])
    pltpu.sync_copy(accl_s.at[sl], aggl_hbm.at[c, sl])


_mid_call = pl.kernel(
    _mid_body,
    out_type=[jax.ShapeDtypeStruct((_NC, _NPAD, 128), _f32),
              jax.ShapeDtypeStruct((_NC, _NPAD), _f32),
              jax.ShapeDtypeStruct((_NC, _NPAD, 16), _f32)],
    mesh=_sc_mesh,
    compiler_params=_sc_params,
    scratch_types=[
        pltpu.VMEM((_NCHUNK, _CH), jnp.int32),
        pltpu.VMEM((_NCHUNK, _CH), jnp.int32),
        pltpu.VMEM((_CH, 128), _f32),
        pltpu.VMEM((_CH, 16), _f32),
        pltpu.VMEM((_CH,), _f32),
        pltpu.VMEM_SHARED((_NPAD, 128), _f32),
        pltpu.VMEM_SHARED((_NPAD,), _f32),
        pltpu.VMEM_SHARED((_NPAD, 16), _f32),
        pltpu.SemaphoreType.DMA,
    ])


def _l2_body(y2_hbm, srcl_hbm, dstl_hbm, agg_hbm,
             sidx, didx, gbuf, acc_s, sem):
    c = lax.axis_index("c")
    s = lax.axis_index("s")
    _fill_2d(gbuf, _CH, 128, 0.0)
    for k in range(_RPT // _CH):
        pltpu.sync_copy(gbuf, acc_s.at[pl.ds(s * _RPT + k * _CH, _CH)])
    plsc.subcore_barrier()
    _spmv_loop(y2_hbm, srcl_hbm, dstl_hbm, c, s, sidx, didx, gbuf, acc_s,
               sem)
    plsc.subcore_barrier()
    sl = pl.ds(s * _RPT, _RPT)
    pltpu.sync_copy(acc_s.at[sl], agg_hbm.at[c, sl])


_l2_call = pl.kernel(
    _l2_body,
    out_type=[jax.ShapeDtypeStruct((_NC, _NPAD, 128), _f32)],
    mesh=_sc_mesh,
    compiler_params=_sc_params,
    scratch_types=[
        pltpu.VMEM((_NCHUNK, _CH), jnp.int32),
        pltpu.VMEM((_NCHUNK, _CH), jnp.int32),
        pltpu.VMEM((_CH, 128), _f32),
        pltpu.VMEM_SHARED((_NPAD, 128), _f32),
        pltpu.SemaphoreType.DMA,
    ])


def _prep_body(degp_ref, degl_ref, xp_ref, xl_ref,
               yp_ref, yl_ref, invp_ref, invl_ref):
    invp = lax.rsqrt(degp_ref[...])
    invl = lax.rsqrt(degl_ref[...])
    invp_ref[...] = invp
    invl_ref[...] = invl
    yp_ref[...] = xp_ref[...] * invp
    yl_ref[...] = xl_ref[...] * invl


def _prep_call(degp_col, degl_col, xp_pad, xl_pad):
    return pl.pallas_call(
        _prep_body,
        grid=(_GRID,),
        in_specs=[pl.BlockSpec((_BM, 1), lambda i: (i, 0)),
                  pl.BlockSpec((_BM, 1), lambda i: (i, 0)),
                  pl.BlockSpec((_BM, 128), lambda i: (i, 0)),
                  pl.BlockSpec((_BM, 16), lambda i: (i, 0))],
        out_specs=[pl.BlockSpec((_BM, 128), lambda i: (i, 0)),
                   pl.BlockSpec((_BM, 16), lambda i: (i, 0)),
                   pl.BlockSpec((_BM, 1), lambda i: (i, 0)),
                   pl.BlockSpec((_BM, 1), lambda i: (i, 0))],
        out_shape=[jax.ShapeDtypeStruct((_NPAD, 128), _f32),
                   jax.ShapeDtypeStruct((_NPAD, 16), _f32),
                   jax.ShapeDtypeStruct((_NPAD, 1), _f32),
                   jax.ShapeDtypeStruct((_NPAD, 1), _f32)],
    )(degp_col, degl_col, xp_pad, xl_pad)


def _mid_tc_body(p0_ref, p1_ref, yp_ref, invp_ref, g_ref,
                 l0_ref, l1_ref, yl_ref, invl_ref,
                 wp1_ref, bp1_ref, wp2_ref, bp2_ref, wl1_ref, bl1_ref,
                 y2_ref, zp_ref, sacc_ref):
    i = pl.program_id(0)
    invp = invp_ref[...]
    aggp = invp * (p0_ref[...] + p1_ref[...] + yp_ref[...])
    h = jnp.maximum(
        jnp.dot(aggp, wp1_ref[...], preferred_element_type=_f32)
        + bp1_ref[...], 0.0)
    rowid = lax.broadcasted_iota(jnp.int32, (_BM, 1), 0) + i * _BM
    cvec = jnp.where(rowid < _N, invp * (g_ref[...] + invp), 0.0)

    @pl.when(i == 0)
    def _():
        sacc_ref[...] = jnp.zeros_like(sacc_ref)

    sacc_ref[...] += jnp.sum(cvec * h, axis=0, keepdims=True)

    invl = invl_ref[...]
    aggl = invl * (l0_ref[...] + l1_ref[...] + yl_ref[...])
    hl = jnp.maximum(
        jnp.dot(aggl, wl1_ref[...], preferred_element_type=_f32)
        + bl1_ref[...], 0.0)
    y2_ref[...] = invl * hl

    @pl.when(i == _GRID - 1)
    def _():
        zp_ref[...] = jnp.dot(sacc_ref[...] / _N, wp2_ref[...],
                              preferred_element_type=_f32) + bp2_ref[...]


def _mid_tc_call(p0, p1, yp, invp, gcol, l0, l1, yl, invl,
                 Wp1, bp1, Wp2, bp2, Wl1p, bl1):
    blk = lambda r, c: pl.BlockSpec((r, c), lambda i: (i, 0))
    cst = lambda r, c: pl.BlockSpec((r, c), lambda i: (0, 0))
    return pl.pallas_call(
        _mid_tc_body,
        grid=(_GRID,),
        in_specs=[blk(_BM, 128), blk(_BM, 128), blk(_BM, 128), blk(_BM, 1),
                  blk(_BM, 1),
                  blk(_BM, 16), blk(_BM, 16), blk(_BM, 16), blk(_BM, 1),
                  cst(128, 128), cst(1, 128), cst(128, 128), cst(1, 128),
                  cst(16, 128), cst(1, 128)],
        out_specs=[blk(_BM, 128), cst(1, 128)],
        out_shape=[jax.ShapeDtypeStruct((_NPAD, 128), _f32),
                   jax.ShapeDtypeStruct((1, 128), _f32)],
        scratch_shapes=[pltpu.VMEM((1, 128), _f32)],
    )(p0, p1, yp, invp, gcol, l0, l1, yl, invl,
      Wp1, bp1[None, :], Wp2, bp2[None, :], Wl1p, bl1[None, :])


def _fin_body(bfs_ref, q0_ref, q1_ref, y2_ref, invl_ref, lab_ref,
              labf_ref, wl2_ref, bl2_ref, wf_ref, bf_ref,
              wd1_ref, bd1_ref, wd2_ref, bd2_ref,
              zv_ref, lp_ref, zlsum_ref, zvsum_ref, labA_ref, labB_ref):
    i = pl.program_id(0)
    rowid = lax.broadcasted_iota(jnp.int32, (_BM, 1), 0) + i * _BM

    @pl.when(i == 0)
    def _():
        lp_ref[...] = jnp.zeros_like(lp_ref)
        zlsum_ref[...] = jnp.zeros_like(zlsum_ref)
        zvsum_ref[...] = jnp.zeros_like(zvsum_ref)
        labA_ref[...] = jnp.zeros_like(labA_ref)
        labB_ref[...] = jnp.zeros_like(labB_ref)

    invl = invl_ref[...]
    aggl2 = invl * (q0_ref[...] + q1_ref[...] + y2_ref[...])
    zl = jnp.dot(aggl2, wl2_ref[...], preferred_element_type=_f32) \
        + bl2_ref[...]
    logits = jnp.dot(zl, wf_ref[...], preferred_element_type=_f32) \
        + bf_ref[...]
    m = jnp.max(logits, axis=1, keepdims=True)
    e = jnp.exp(logits - m)
    lab = lab_ref[...]
    num = jnp.sum(e * lab, axis=1, keepdims=True)
    den = jnp.sum(e, axis=1, keepdims=True)
    lig_mask = rowid < _N
    inner = jnp.where(lig_mask, num / den, 1.0)
    lp_ref[...] += jnp.sum(jnp.log(inner), axis=0, keepdims=True)
    zlsum_ref[...] += jnp.sum(jnp.where(lig_mask, zl, 0.0), axis=0,
                              keepdims=True)
    labA_ref[...] += jnp.sum(jnp.where(lig_mask, lab, 0.0), axis=0,
                             keepdims=True)
    labB_ref[...] += jnp.sum(jnp.where(rowid < _ND, lab, 0.0), axis=0,
                             keepdims=True)

    src0 = bfs_ref[0, 0]
    dst0 = bfs_ref[1, 0]
    same = src0 == dst0
    isq = 0.70710678118654752
    x_s0 = labf_ref[pl.ds(src0, 1), :]
    x_d0 = labf_ref[pl.ds(dst0, 1), :]
    agg_d = jnp.where(same, x_d0, isq * x_s0 + 0.5 * x_d0)
    h_spec = jnp.maximum(
        jnp.dot(agg_d, wd1_ref[...], preferred_element_type=_f32)
        + bd1_ref[...], 0.0)
    h_src0 = jnp.maximum(
        jnp.dot(x_s0, wd1_ref[...], preferred_element_type=_f32)
        + bd1_ref[...], 0.0)
    agg2_d = jnp.where(same, h_spec, isq * h_src0 + 0.5 * h_spec)
    z_spec = jnp.dot(agg2_d, wd2_ref[...], preferred_element_type=_f32) \
        + bd2_ref[...]
    h = jnp.maximum(
        jnp.dot(lab, wd1_ref[...], preferred_element_type=_f32)
        + bd1_ref[...], 0.0)
    h = jnp.where(rowid == dst0, h_spec, h)
    zv = jnp.dot(h, wd2_ref[...], preferred_element_type=_f32) \
        + bd2_ref[...]
    zv = jnp.where(rowid == dst0, z_spec, zv)
    zv_ref[...] = zv
    zvsum_ref[...] += jnp.sum(jnp.where(rowid < _ND, zv, 0.0), axis=0,
                              keepdims=True)


def _fin_call(bfs, q0, q1, y2, invl, lab16, Wl2, bl2, Wf16, bf16,
              Wd1p, bd1, Wd2, bd2):
    blk = lambda r, c: pl.BlockSpec((r, c), lambda i: (i, 0))
    cst = lambda r, c: pl.BlockSpec((r, c), lambda i: (0, 0))
    return pl.pallas_call(
        _fin_body,
        grid=(_GRID,),
        in_specs=[pl.BlockSpec(memory_space=pltpu.SMEM),
                  blk(_BM, 128), blk(_BM, 128), blk(_BM, 128), blk(_BM, 1),
                  blk(_BM, 16),
                  cst(_NPAD, 16),
                  cst(128, 128), cst(1, 128), cst(128, 16), cst(1, 16),
                  cst(16, 128), cst(1, 128), cst(128, 128), cst(1, 128)],
        out_specs=[blk(_BM, 128), cst(1, 1), cst(1, 128), cst(1, 128),
                   cst(1, 16), cst(1, 16)],
        out_shape=[jax.ShapeDtypeStruct((_NPAD, 128), _f32),
                   jax.ShapeDtypeStruct((1, 1), _f32),
                   jax.ShapeDtypeStruct((1, 128), _f32),
                   jax.ShapeDtypeStruct((1, 128), _f32),
                   jax.ShapeDtypeStruct((1, 16), _f32),
                   jax.ShapeDtypeStruct((1, 16), _f32)],
    )(bfs, q0, q1, y2, invl, lab16, lab16, Wl2, bl2[None, :], Wf16,
      bf16[None, :], Wd1p, bd1[None, :], Wd2, bd2[None, :])


def kernel(x_p, edge_index_p, x_l, edge_index_l, bfs_init, Wp1, bp1, Wp2,
           bp2, Wl1, bl1, Wl2, bl2, Wd1, bd1, Wd2, bd2, Wf, bf):
    srcp3 = edge_index_p[0].reshape(_NC, _NS, _NCHUNK, _CH)
    dstp3 = edge_index_p[1].reshape(_NC, _NS, _NCHUNK, _CH)
    srcl3 = edge_index_l[0].reshape(_NC, _NS, _NCHUNK, _CH)
    dstl3 = edge_index_l[1].reshape(_NC, _NS, _NCHUNK, _CH)

    dp, dl = _deg_call(dstp3, dstl3)
    degp_col = (dp[0] + dp[1] + 1.0)[:, None]
    degl_col = (dl[0] + dl[1] + 1.0)[:, None]

    xp_pad = jnp.pad(x_p, ((0, _NPAD - _N), (0, 0)))
    xl_pad = jnp.pad(x_l, ((0, _NPAD - _N), (0, 1)))
    yp, yl, invp, invl = _prep_call(degp_col, degl_col, xp_pad, xl_pad)

    aggp, gpart, aggl = _mid_call(yp, srcp3, dstp3,
                                  invp.reshape(_NPAD), yl, srcl3, dstl3)
    gcol = (gpart[0] + gpart[1])[:, None]

    Wl1p = jnp.pad(Wl1, ((0, 1), (0, 0)))
    y2, zp = _mid_tc_call(aggp[0], aggp[1], yp, invp, gcol,
                          aggl[0], aggl[1], yl, invl,
                          Wp1, bp1, Wp2, bp2, Wl1p, bl1)
    z_pocket = zp[0]

    (aggl2,) = _l2_call(y2, srcl3, dstl3)

    lab_v = x_l[:, 4:]
    stop_row = jnp.zeros((1, 16), _f32).at[0, _NATOM - 1].set(1.0)
    lab16 = jnp.concatenate([
        jnp.pad(lab_v, ((0, 0), (0, 5))), stop_row,
        jnp.zeros((_NPAD - _ND, 16), _f32)], axis=0)
    Wf16 = jnp.pad(Wf, ((0, 0), (0, 5)))
    bf16 = jnp.concatenate([bf[:_NATOM - 1], bf[_NATOM - 1:] - 1e9,
                            jnp.full((5,), -1e9, _f32)])
    Wd1p = jnp.pad(Wd1, ((0, 5), (0, 0)))
    zv, lp, zlsum, zvsum, labA, labB = _fin_call(
        bfs_init, aggl2[0], aggl2[1], y2, invl, lab16,
        Wl2, bl2, Wf16, bf16, Wd1p, bd1, Wd2, bd2)

    log_prob = lp[0, 0]
    z_v = zv[:_ND]
    H_init = jnp.concatenate([zlsum[0] / _N, labA[0, :_NATOM] / _N])
    H_t = jnp.concatenate([zvsum[0] / _ND, labB[0, :_NATOM] / _ND])
    return (log_prob, z_pocket, z_v, H_init, H_t)


# speedup vs baseline: 29.6875x; 1.3207x over previous
import functools

import jax
import jax.numpy as jnp
from jax import lax
from jax.experimental import pallas as pl
from jax.experimental.pallas import tpu as pltpu
from jax.experimental.pallas import tpu_sc as plsc

_N = 10000
_E = 320000
_NPAD = 10240
_NATOM = 11
_ND = 10001

_NC, _NS, _LANES = 2, 16, 16
_CH = 80
_NCHL = _E // (_NC * _NS * _CH)
_NCHP = _E // (_NS * _CH)
_RPT = _NPAD // _NS
_HD = 64

_BM = 512
_GRID = _NPAD // _BM

_f32 = jnp.float32

_sc_mesh = plsc.VectorSubcoreMesh(
    core_axis_name="c", subcore_axis_name="s",
    num_cores=_NC, num_subcores=_NS)
_sc_params = pltpu.CompilerParams(use_tc_tiling_on_sc=False)


def _fill_1d(buf, n, value):
    v = jnp.full((_LANES,), value, _f32)

    def body(i, _):
        buf[pl.ds(i * _LANES, _LANES)] = v
        return 0

    lax.fori_loop(0, n // _LANES, body, 0)


def _fill_2d(buf, rows, cols, value):
    v = jnp.full((_LANES,), value, _f32)
    nseg = cols // _LANES

    def body(i, _):
        for k in range(nseg):
            buf[i, pl.ds(k * _LANES, _LANES)] = v
        return 0

    lax.fori_loop(0, rows, body, 0)


def _db_loop(nchunks, base, start_fn, scatter_fn, buf0, buf1, sem0, sem1,
             wait_fn):
    start_fn(buf0, sem0, base)
    start_fn(buf1, sem1, base + 1)

    def body(t, _):
        j = base + 2 * t
        wait_fn(buf0, sem0)
        scatter_fn(buf0, j)

        @pl.when(j + 2 < base + nchunks)
        def _():
            start_fn(buf0, sem0, j + 2)

        wait_fn(buf1, sem1)
        scatter_fn(buf1, j + 1)

        @pl.when(j + 3 < base + nchunks)
        def _():
            start_fn(buf1, sem1, j + 3)

        return 0

    lax.fori_loop(0, nchunks // 2, body, 0)
    if nchunks % 2:
        wait_fn(buf0, sem0)
        scatter_fn(buf0, base + nchunks - 1)


def _deg_body(dstp_hbm, dstl_hbm, outp_hbm, outl_hbm,
              idx_v, ones_v, zed_v, histp_s, histl_s, sem):
    c = lax.axis_index("c")
    s = lax.axis_index("s")
    _fill_1d(zed_v, _RPT, 0.0)
    _fill_1d(ones_v, _CH, 1.0)
    pltpu.sync_copy(zed_v, histp_s.at[pl.ds(s * _RPT, _RPT)])
    pltpu.sync_copy(zed_v, histl_s.at[pl.ds(s * _RPT, _RPT)])
    plsc.subcore_barrier()

    def scatter_ones(hist_s):
        def wait_one():
            pltpu.make_async_copy(ones_v, hist_s.at[idx_v.at[0]],
                                  sem).wait()

        def body(j, _):
            @pl.when(j >= 5)
            def _():
                wait_one()

            pltpu.async_copy(ones_v, hist_s.at[idx_v.at[j]], sem, add=True)
            return 0

        lax.fori_loop(0, _NCHL, body, 0)
        for _k in range(5):
            wait_one()

    pltpu.sync_copy(dstp_hbm.at[c, s], idx_v)
    scatter_ones(histp_s)
    pltpu.sync_copy(dstl_hbm.at[c, s], idx_v)
    scatter_ones(histl_s)
    plsc.subcore_barrier()
    sl = pl.ds(s * _RPT, _RPT)
    pltpu.sync_copy(histp_s.at[sl], outp_hbm.at[c, sl])
    pltpu.sync_copy(histl_s.at[sl], outl_hbm.at[c, sl])


_deg_call = pl.kernel(
    _deg_body,
    out_type=[jax.ShapeDtypeStruct((_NC, _NPAD), _f32),
              jax.ShapeDtypeStruct((_NC, _NPAD), _f32)],
    mesh=_sc_mesh,
    compiler_params=_sc_params,
    scratch_types=[
        pltpu.VMEM((_NCHL, _CH), jnp.int32),
        pltpu.VMEM((_CH,), _f32),
        pltpu.VMEM((_RPT,), _f32),
        pltpu.VMEM_SHARED((_NPAD,), _f32),
        pltpu.VMEM_SHARED((_NPAD,), _f32),
        pltpu.SemaphoreType.DMA,
    ])


def _zero_acc2d(gbuf, acc_s, s, cols):
    _fill_2d(gbuf, _CH, cols, 0.0)
    for k in range(_RPT // _CH):
        pltpu.sync_copy(gbuf, acc_s.at[pl.ds(s * _RPT + k * _CH, _CH)])


def _row_spmv(y_hbm, nch, base, sidx, didx, b0, b1, sem0, sem1, acc_s, cols):
    def start(buf, sem, j):
        pltpu.async_copy(y_hbm.at[sidx.at[j]], buf, sem)

    def wait(buf, sem):
        pltpu.make_async_copy(y_hbm.at[pl.ds(0, _CH)], buf, sem).wait()

    def scat(buf, j):
        pltpu.sync_copy(buf, acc_s.at[didx.at[j]], add=True)

    _db_loop(nch, base, start, scat, b0, b1, sem0, sem1, wait)


def _mid_body(yp_hbm, srcp_hbm, dstp_hbm, inv_hbm, yl_hbm, srcl_hbm,
              dstl_hbm, aggp_hbm, g_hbm, aggl_hbm,
              sidx, didx, gb0, gb1, vb0, vb1, lb0, lb1,
              accp_s, gacc_s, accl_s, sem0, sem1, sem2, sem3):
    c = lax.axis_index("c")
    s = lax.axis_index("s")
    _zero_acc2d(gb0, accp_s, s, _HD)
    _zero_acc2d(lb0, accl_s, s, 16)
    _fill_1d(vb0, _CH, 0.0)
    for k in range(_RPT // _CH):
        pltpu.sync_copy(vb0, gacc_s.at[pl.ds(s * _RPT + k * _CH, _CH)])
    plsc.subcore_barrier()

    pltpu.sync_copy(srcp_hbm.at[s], sidx)
    pltpu.sync_copy(dstp_hbm.at[s], didx)
    _row_spmv(yp_hbm.at[c], _NCHP, 0, sidx, didx, gb0, gb1, sem0, sem1,
              accp_s, _HD)

    def gstart(buf, sem, j):
        pltpu.async_copy(inv_hbm.at[didx.at[j]], buf, sem)

    def gwait(buf, sem):
        pltpu.make_async_copy(inv_hbm.at[pl.ds(0, _CH)], buf, sem).wait()

    def gscat(buf, j):
        pltpu.sync_copy(buf, gacc_s.at[sidx.at[j]], add=True)

    _db_loop(_NCHL, c * _NCHL, gstart, gscat, vb0, vb1, sem2, sem3, gwait)

    pltpu.sync_copy(srcl_hbm.at[c, s], sidx.at[pl.ds(0, _NCHL)])
    pltpu.sync_copy(dstl_hbm.at[c, s], didx.at[pl.ds(0, _NCHL)])
    _row_spmv(yl_hbm, _NCHL, 0, sidx, didx, lb0, lb1, sem0, sem1,
              accl_s, 16)

    plsc.subcore_barrier()
    sl = pl.ds(s * _RPT, _RPT)
    pltpu.sync_copy(accp_s.at[sl], aggp_hbm.at[c, sl])
    pltpu.sync_copy(gacc_s.at[sl], g_hbm.at[c, sl])
    pltpu.sync_copy(accl_s.at[sl], aggl_hbm.at[c, sl])


_mid_call = pl.kernel(
    _mid_body,
    out_type=[jax.ShapeDtypeStruct((_NC, _NPAD, _HD), _f32),
              jax.ShapeDtypeStruct((_NC, _NPAD), _f32),
              jax.ShapeDtypeStruct((_NC, _NPAD, 16), _f32)],
    mesh=_sc_mesh,
    compiler_params=_sc_params,
    scratch_types=[
        pltpu.VMEM((_NCHP, _CH), jnp.int32),
        pltpu.VMEM((_NCHP, _CH), jnp.int32),
        pltpu.VMEM((_CH, _HD), _f32),
        pltpu.VMEM((_CH, _HD), _f32),
        pltpu.VMEM((_CH,), _f32),
        pltpu.VMEM((_CH,), _f32),
        pltpu.VMEM((_CH, 16), _f32),
        pltpu.VMEM((_CH, 16), _f32),
        pltpu.VMEM_SHARED((_NPAD, _HD), _f32),
        pltpu.VMEM_SHARED((_NPAD,), _f32),
        pltpu.VMEM_SHARED((_NPAD, 16), _f32),
        pltpu.SemaphoreType.DMA,
        pltpu.SemaphoreType.DMA,
        pltpu.SemaphoreType.DMA,
        pltpu.SemaphoreType.DMA,
    ])


def _l2_body(y2_hbm, srcl_hbm, dstl_hbm, agg_hbm,
             sidx, didx, gb0, gb1, acc_s, sem0, sem1):
    c = lax.axis_index("c")
    s = lax.axis_index("s")
    _zero_acc2d(gb0, acc_s, s, _HD)
    plsc.subcore_barrier()
    pltpu.sync_copy(srcl_hbm.at[s], sidx)
    pltpu.sync_copy(dstl_hbm.at[s], didx)
    _row_spmv(y2_hbm.at[c], _NCHP, 0, sidx, didx, gb0, gb1, sem0, sem1,
              acc_s, _HD)
    plsc.subcore_barrier()
    sl = pl.ds(s * _RPT, _RPT)
    pltpu.sync_copy(acc_s.at[sl], agg_hbm.at[c, sl])


_l2_call = pl.kernel(
    _l2_body,
    out_type=[jax.ShapeDtypeStruct((_NC, _NPAD, _HD), _f32)],
    mesh=_sc_mesh,
    compiler_params=_sc_params,
    scratch_types=[
        pltpu.VMEM((_NCHP, _CH), jnp.int32),
        pltpu.VMEM((_NCHP, _CH), jnp.int32),
        pltpu.VMEM((_CH, _HD), _f32),
        pltpu.VMEM((_CH, _HD), _f32),
        pltpu.VMEM_SHARED((_NPAD, _HD), _f32),
        pltpu.SemaphoreType.DMA,
        pltpu.SemaphoreType.DMA,
    ])


def _prep_body(degp_ref, degl_ref, xp_ref, xl_ref,
               yp_ref, yl_ref, invp_ref, invl_ref):
    invp = lax.rsqrt(degp_ref[...])
    invl = lax.rsqrt(degl_ref[...])
    invp_ref[...] = invp
    invl_ref[...] = invl
    yp = xp_ref[...] * invp
    yp_ref[0, :, :] = yp[:, :_HD]
    yp_ref[1, :, :] = yp[:, _HD:]
    yl_ref[...] = xl_ref[...] * invl


def _prep_call(degp_col, degl_col, xp_pad, xl_pad):
    return pl.pallas_call(
        _prep_body,
        grid=(_GRID,),
        in_specs=[pl.BlockSpec((_BM, 1), lambda i: (i, 0)),
                  pl.BlockSpec((_BM, 1), lambda i: (i, 0)),
                  pl.BlockSpec((_BM, 128), lambda i: (i, 0)),
                  pl.BlockSpec((_BM, 16), lambda i: (i, 0))],
        out_specs=[pl.BlockSpec((_NC, _BM, _HD), lambda i: (0, i, 0)),
                   pl.BlockSpec((_BM, 16), lambda i: (i, 0)),
                   pl.BlockSpec((_BM, 1), lambda i: (i, 0)),
                   pl.BlockSpec((_BM, 1), lambda i: (i, 0))],
        out_shape=[jax.ShapeDtypeStruct((_NC, _NPAD, _HD), _f32),
                   jax.ShapeDtypeStruct((_NPAD, 16), _f32),
                   jax.ShapeDtypeStruct((_NPAD, 1), _f32),
                   jax.ShapeDtypeStruct((_NPAD, 1), _f32)],
    )(degp_col, degl_col, xp_pad, xl_pad)


def _mid_tc_body(ap_ref, yp_ref, invp_ref, g_ref,
                 al_ref, yl_ref, invl_ref,
                 wp1_ref, bp1_ref, wp2_ref, bp2_ref, wl1_ref, bl1_ref,
                 y2_ref, zp_ref, sacc_ref):
    i = pl.program_id(0)
    invp = invp_ref[...]
    aggp = invp * jnp.concatenate(
        [ap_ref[0] + yp_ref[0], ap_ref[1] + yp_ref[1]], axis=1)
    h = jnp.maximum(
        jnp.dot(aggp, wp1_ref[...], preferred_element_type=_f32)
        + bp1_ref[...], 0.0)
    rowid = lax.broadcasted_iota(jnp.int32, (_BM, 1), 0) + i * _BM
    cvec = jnp.where(rowid < _N, invp * (g_ref[...] + invp), 0.0)

    @pl.when(i == 0)
    def _():
        sacc_ref[...] = jnp.zeros_like(sacc_ref)

    sacc_ref[...] += jnp.sum(cvec * h, axis=0, keepdims=True)

    invl = invl_ref[...]
    aggl = invl * (al_ref[0] + al_ref[1] + yl_ref[...])
    hl = jnp.maximum(
        jnp.dot(aggl, wl1_ref[...], preferred_element_type=_f32)
        + bl1_ref[...], 0.0)
    y2 = invl * hl
    y2_ref[0, :, :] = y2[:, :_HD]
    y2_ref[1, :, :] = y2[:, _HD:]

    @pl.when(i == _GRID - 1)
    def _():
        zp_ref[...] = jnp.dot(sacc_ref[...] / _N, wp2_ref[...],
                              preferred_element_type=_f32) + bp2_ref[...]


def _mid_tc_call(aggp, yp3, invp, gcol, aggl, yl, invl,
                 Wp1, bp1, Wp2, bp2, Wl1p, bl1):
    blk = lambda r, c: pl.BlockSpec((r, c), lambda i: (i, 0))
    blk3 = lambda c: pl.BlockSpec((_NC, _BM, c), lambda i: (0, i, 0))
    cst = lambda r, c: pl.BlockSpec((r, c), lambda i: (0, 0))
    return pl.pallas_call(
        _mid_tc_body,
        grid=(_GRID,),
        in_specs=[blk3(_HD), blk3(_HD), blk(_BM, 1), blk(_BM, 1),
                  blk3(16), blk(_BM, 16), blk(_BM, 1),
                  cst(128, 128), cst(1, 128), cst(128, 128), cst(1, 128),
                  cst(16, 128), cst(1, 128)],
        out_specs=[blk3(_HD), cst(1, 128)],
        out_shape=[jax.ShapeDtypeStruct((_NC, _NPAD, _HD), _f32),
                   jax.ShapeDtypeStruct((1, 128), _f32)],
        scratch_shapes=[pltpu.VMEM((1, 128), _f32)],
    )(aggp, yp3, invp, gcol, aggl, yl, invl,
      Wp1, bp1[None, :], Wp2, bp2[None, :], Wl1p, bl1[None, :])


def _fin_body(bfs_ref, q_ref, y2_ref, invl_ref, lab_ref,
              labf_ref, wl2_ref, bl2_ref, wf_ref, bf_ref,
              wd1_ref, bd1_ref, wd2_ref, bd2_ref,
              zv_ref, lp_ref, zlsum_ref, zvsum_ref, labA_ref, labB_ref):
    i = pl.program_id(0)
    rowid = lax.broadcasted_iota(jnp.int32, (_BM, 1), 0) + i * _BM

    @pl.when(i == 0)
    def _():
        lp_ref[...] = jnp.zeros_like(lp_ref)
        zlsum_ref[...] = jnp.zeros_like(zlsum_ref)
        zvsum_ref[...] = jnp.zeros_like(zvsum_ref)
        labA_ref[...] = jnp.zeros_like(labA_ref)
        labB_ref[...] = jnp.zeros_like(labB_ref)

    invl = invl_ref[...]
    aggl2 = invl * jnp.concatenate(
        [q_ref[0] + y2_ref[0], q_ref[1] + y2_ref[1]], axis=1)
    zl = jnp.dot(aggl2, wl2_ref[...], preferred_element_type=_f32) \
        + bl2_ref[...]
    logits = jnp.dot(zl, wf_ref[...], preferred_element_type=_f32) \
        + bf_ref[...]
    m = jnp.max(logits, axis=1, keepdims=True)
    e = jnp.exp(logits - m)
    lab = lab_ref[...]
    num = jnp.sum(e * lab, axis=1, keepdims=True)
    den = jnp.sum(e, axis=1, keepdims=True)
    lig_mask = rowid < _N
    inner = jnp.where(lig_mask, num / den, 1.0)
    lp_ref[...] += jnp.sum(jnp.log(inner), axis=0, keepdims=True)
    zlsum_ref[...] += jnp.sum(jnp.where(lig_mask, zl, 0.0), axis=0,
                              keepdims=True)
    labA_ref[...] += jnp.sum(jnp.where(lig_mask, lab, 0.0), axis=0,
                             keepdims=True)
    labB_ref[...] += jnp.sum(jnp.where(rowid < _ND, lab, 0.0), axis=0,
                             keepdims=True)

    src0 = bfs_ref[0, 0]
    dst0 = bfs_ref[1, 0]
    same = src0 == dst0
    isq = 0.70710678118654752
    x_s0 = labf_ref[pl.ds(src0, 1), :]
    x_d0 = labf_ref[pl.ds(dst0, 1), :]
    agg_d = jnp.where(same, x_d0, isq * x_s0 + 0.5 * x_d0)
    h_spec = jnp.maximum(
        jnp.dot(agg_d, wd1_ref[...], preferred_element_type=_f32)
        + bd1_ref[...], 0.0)
    h_src0 = jnp.maximum(
        jnp.dot(x_s0, wd1_ref[...], preferred_element_type=_f32)
        + bd1_ref[...], 0.0)
    agg2_d = jnp.where(same, h_spec, isq * h_src0 + 0.5 * h_spec)
    z_spec = jnp.dot(agg2_d, wd2_ref[...], preferred_element_type=_f32) \
        + bd2_ref[...]
    h = jnp.maximum(
        jnp.dot(lab, wd1_ref[...], preferred_element_type=_f32)
        + bd1_ref[...], 0.0)
    h = jnp.where(rowid == dst0, h_spec, h)
    zv = jnp.dot(h, wd2_ref[...], preferred_element_type=_f32) \
        + bd2_ref[...]
    zv = jnp.where(rowid == dst0, z_spec, zv)
    zv_ref[...] = zv
    zvsum_ref[...] += jnp.sum(jnp.where(rowid < _ND, zv, 0.0), axis=0,
                              keepdims=True)


def _fin_call(bfs, aggl2, y23, invl, lab16, Wl2, bl2, Wf16, bf16,
              Wd1p, bd1, Wd2, bd2):
    blk = lambda r, c: pl.BlockSpec((r, c), lambda i: (i, 0))
    blk3 = lambda c: pl.BlockSpec((_NC, _BM, c), lambda i: (0, i, 0))
    cst = lambda r, c: pl.BlockSpec((r, c), lambda i: (0, 0))
    return pl.pallas_call(
        _fin_body,
        grid=(_GRID,),
        in_specs=[pl.BlockSpec(memory_space=pltpu.SMEM),
                  blk3(_HD), blk3(_HD), blk(_BM, 1),
                  blk(_BM, 16),
                  cst(_NPAD, 16),
                  cst(128, 128), cst(1, 128), cst(128, 16), cst(1, 16),
                  cst(16, 128), cst(1, 128), cst(128, 128), cst(1, 128)],
        out_specs=[blk(_BM, 128), cst(1, 1), cst(1, 128), cst(1, 128),
                   cst(1, 16), cst(1, 16)],
        out_shape=[jax.ShapeDtypeStruct((_NPAD, 128), _f32),
                   jax.ShapeDtypeStruct((1, 1), _f32),
                   jax.ShapeDtypeStruct((1, 128), _f32),
                   jax.ShapeDtypeStruct((1, 128), _f32),
                   jax.ShapeDtypeStruct((1, 16), _f32),
                   jax.ShapeDtypeStruct((1, 16), _f32)],
    )(bfs, aggl2, y23, invl, lab16, lab16, Wl2, bl2[None, :], Wf16,
      bf16[None, :], Wd1p, bd1[None, :], Wd2, bd2[None, :])


def kernel(x_p, edge_index_p, x_l, edge_index_l, bfs_init, Wp1, bp1, Wp2,
           bp2, Wl1, bl1, Wl2, bl2, Wd1, bd1, Wd2, bd2, Wf, bf):
    srcp16 = edge_index_p[0].reshape(_NS, _NCHP, _CH)
    dstp16 = edge_index_p[1].reshape(_NS, _NCHP, _CH)
    srcl16 = edge_index_l[0].reshape(_NS, _NCHP, _CH)
    dstl16 = edge_index_l[1].reshape(_NS, _NCHP, _CH)
    srcl32 = edge_index_l[0].reshape(_NC, _NS, _NCHL, _CH)
    dstl32 = edge_index_l[1].reshape(_NC, _NS, _NCHL, _CH)
    dstp32 = edge_index_p[1].reshape(_NC, _NS, _NCHL, _CH)

    dp, dl = _deg_call(dstp32, dstl32)
    degp_col = (dp[0] + dp[1] + 1.0)[:, None]
    degl_col = (dl[0] + dl[1] + 1.0)[:, None]

    xp_pad = jnp.pad(x_p, ((0, _NPAD - _N), (0, 0)))
    xl_pad = jnp.pad(x_l, ((0, _NPAD - _N), (0, 1)))
    yp3, yl, invp, invl = _prep_call(degp_col, degl_col, xp_pad, xl_pad)

    aggp, gpart, aggl = _mid_call(yp3, srcp16, dstp16,
                                  invp.reshape(_NPAD), yl, srcl32, dstl32)
    gcol = (gpart[0] + gpart[1])[:, None]

    Wl1p = jnp.pad(Wl1, ((0, 1), (0, 0)))
    y23, zp = _mid_tc_call(aggp, yp3, invp, gcol, aggl, yl, invl,
                           Wp1, bp1, Wp2, bp2, Wl1p, bl1)
    z_pocket = zp[0]

    (aggl2,) = _l2_call(y23, srcl16, dstl16)

    lab_v = x_l[:, 4:]
    stop_row = jnp.zeros((1, 16), _f32).at[0, _NATOM - 1].set(1.0)
    lab16 = jnp.concatenate([
        jnp.pad(lab_v, ((0, 0), (0, 5))), stop_row,
        jnp.zeros((_NPAD - _ND, 16), _f32)], axis=0)
    Wf16 = jnp.pad(Wf, ((0, 0), (0, 5)))
    bf16 = jnp.concatenate([bf[:_NATOM - 1], bf[_NATOM - 1:] - 1e9,
                            jnp.full((5,), -1e9, _f32)])
    Wd1p = jnp.pad(Wd1, ((0, 5), (0, 0)))
    zv, lp, zlsum, zvsum, labA, labB = _fin_call(
        bfs_init, aggl2, y23, invl, lab16,
        Wl2, bl2, Wf16, bf16, Wd1p, bd1, Wd2, bd2)

    log_prob = lp[0, 0]
    z_v = zv[:_ND]
    H_init = jnp.concatenate([zlsum[0] / _N, labA[0, :_NATOM] / _N])
    H_t = jnp.concatenate([zvsum[0] / _ND, labB[0, :_NATOM] / _ND])
    return (log_prob, z_pocket, z_v, H_init, H_t)


# speedup vs baseline: 32.1224x; 1.0820x over previous
import functools

import jax
import jax.numpy as jnp
from jax import lax
from jax.experimental import pallas as pl
from jax.experimental.pallas import tpu as pltpu
from jax.experimental.pallas import tpu_sc as plsc

_N = 10000
_E = 320000
_NPAD = 10240
_NATOM = 11
_ND = 10001

_NC, _NS, _LANES = 2, 16, 16
_CH = 80
_NCHL = _E // (_NC * _NS * _CH)
_NCHP = _E // (_NS * _CH)
_RPT = _NPAD // _NS
_HD = 64

_BM = 512
_GRID = _NPAD // _BM

_f32 = jnp.float32

_sc_mesh = plsc.VectorSubcoreMesh(
    core_axis_name="c", subcore_axis_name="s",
    num_cores=_NC, num_subcores=_NS)
_sc_params = pltpu.CompilerParams(use_tc_tiling_on_sc=False)


def _fill_1d(buf, n, value):
    v = jnp.full((_LANES,), value, _f32)

    def body(i, _):
        buf[pl.ds(i * _LANES, _LANES)] = v
        return 0

    lax.fori_loop(0, n // _LANES, body, 0)


def _fill_2d(buf, rows, cols, value):
    v = jnp.full((_LANES,), value, _f32)
    nseg = cols // _LANES

    def body(i, _):
        for k in range(nseg):
            buf[i, pl.ds(k * _LANES, _LANES)] = v
        return 0

    lax.fori_loop(0, rows, body, 0)


class _Stream:
    def __init__(self, y_hbm, gidx, sidx2, acc_s, b0, b1, gs0, gs1,
                 ss0, ss1):
        self.y_hbm = y_hbm
        self.gidx = gidx
        self.sidx = sidx2
        self.acc = acc_s
        self.b = (b0, b1)
        self.gs = (gs0, gs1)
        self.ss = (ss0, ss1)

    def start(self, k, j):
        pltpu.async_copy(self.y_hbm.at[self.gidx.at[j]], self.b[k],
                         self.gs[k])

    def gwait(self, k):
        pltpu.make_async_copy(self.y_hbm.at[pl.ds(0, _CH)], self.b[k],
                              self.gs[k]).wait()

    def scat(self, k, j):
        pltpu.async_copy(self.b[k], self.acc.at[self.sidx.at[j]],
                         self.ss[k], add=True)

    def swait(self, k):
        pltpu.make_async_copy(self.b[k], self.acc.at[self.sidx.at[0]],
                              self.ss[k]).wait()


def _run_stream(st, nch, base):
    st.start(0, base)
    st.start(1, base + 1)

    def body(t, _):
        j = base + 2 * t
        st.gwait(0)
        st.scat(0, j)
        st.gwait(1)
        st.scat(1, j + 1)

        @pl.when(j + 2 < base + nch)
        def _():
            st.swait(0)
            st.start(0, j + 2)

        @pl.when(j + 3 < base + nch)
        def _():
            st.swait(1)
            st.start(1, j + 3)

        return 0

    lax.fori_loop(0, nch // 2, body, 0)
    if nch % 2:
        st.gwait(0)
        st.scat(0, base + nch - 1)
    st.swait(0)
    st.swait(1)


def _deg_body(dstp_hbm, dstl_hbm, outp_hbm, outl_hbm,
              idx_v, ones_v, zed_v, histp_s, histl_s, sem):
    c = lax.axis_index("c")
    s = lax.axis_index("s")
    _fill_1d(zed_v, _RPT, 0.0)
    _fill_1d(ones_v, _CH, 1.0)
    pltpu.sync_copy(zed_v, histp_s.at[pl.ds(s * _RPT, _RPT)])
    pltpu.sync_copy(zed_v, histl_s.at[pl.ds(s * _RPT, _RPT)])
    plsc.subcore_barrier()

    def scatter_ones(hist_s):
        def wait_one():
            pltpu.make_async_copy(ones_v, hist_s.at[idx_v.at[0]],
                                  sem).wait()

        def body(j, _):
            @pl.when(j >= 5)
            def _():
                wait_one()

            pltpu.async_copy(ones_v, hist_s.at[idx_v.at[j]], sem, add=True)
            return 0

        lax.fori_loop(0, _NCHL, body, 0)
        for _k in range(5):
            wait_one()

    pltpu.sync_copy(dstp_hbm.at[c, s], idx_v)
    scatter_ones(histp_s)
    pltpu.sync_copy(dstl_hbm.at[c, s], idx_v)
    scatter_ones(histl_s)
    plsc.subcore_barrier()
    sl = pl.ds(s * _RPT, _RPT)
    pltpu.sync_copy(histp_s.at[sl], outp_hbm.at[c, sl])
    pltpu.sync_copy(histl_s.at[sl], outl_hbm.at[c, sl])


_deg_call = pl.kernel(
    _deg_body,
    out_type=[jax.ShapeDtypeStruct((_NC, _NPAD), _f32),
              jax.ShapeDtypeStruct((_NC, _NPAD), _f32)],
    mesh=_sc_mesh,
    compiler_params=_sc_params,
    scratch_types=[
        pltpu.VMEM((_NCHL, _CH), jnp.int32),
        pltpu.VMEM((_CH,), _f32),
        pltpu.VMEM((_RPT,), _f32),
        pltpu.VMEM_SHARED((_NPAD,), _f32),
        pltpu.VMEM_SHARED((_NPAD,), _f32),
        pltpu.SemaphoreType.DMA,
    ])


def _zero_acc2d(gbuf, acc_s, s, cols):
    _fill_2d(gbuf, _CH, cols, 0.0)
    for k in range(_RPT // _CH):
        pltpu.sync_copy(gbuf, acc_s.at[pl.ds(s * _RPT + k * _CH, _CH)])


def _mid_body(yp_hbm, srcp_hbm, dstp_hbm, inv_hbm, yl_hbm, srcl_hbm,
              dstl_hbm, aggp_hbm, g_hbm, aggl_hbm,
              sidx, didx, lsidx, ldidx, rb0, rb1, vb0, vb1, lb0, lb1,
              accp_s, gacc_s, accl_s,
              rgs0, rgs1, rss0, rss1, ggs0, ggs1, gss0, gss1,
              lgs0, lgs1, lss0, lss1):
    c = lax.axis_index("c")
    s = lax.axis_index("s")
    _zero_acc2d(rb0, accp_s, s, _HD)
    _zero_acc2d(lb0, accl_s, s, 16)
    _fill_1d(vb0, _CH, 0.0)
    for k in range(_RPT // _CH):
        pltpu.sync_copy(vb0, gacc_s.at[pl.ds(s * _RPT + k * _CH, _CH)])
    plsc.subcore_barrier()

    pltpu.sync_copy(srcp_hbm.at[s], sidx)
    pltpu.sync_copy(dstp_hbm.at[s], didx)
    pltpu.sync_copy(srcl_hbm.at[c, s], lsidx)
    pltpu.sync_copy(dstl_hbm.at[c, s], ldidx)

    R = _Stream(yp_hbm.at[c], sidx, didx, accp_s, rb0, rb1,
                rgs0, rgs1, rss0, rss1)
    G = _Stream(inv_hbm, didx, sidx, gacc_s, vb0, vb1,
                ggs0, ggs1, gss0, gss1)
    L = _Stream(yl_hbm, lsidx, ldidx, accl_s, lb0, lb1,
                lgs0, lgs1, lss0, lss1)

    gb = c * _NCHL
    R.start(0, 0)
    R.start(1, 1)
    G.start(0, gb)
    G.start(1, gb + 1)
    L.start(0, 0)
    L.start(1, 1)
    nhalf = _NCHL // 2

    def body(t, _):
        j = 2 * t
        R.gwait(0)
        R.scat(0, j)
        R.gwait(1)
        R.scat(1, j + 1)

        @pl.when(j + 2 < _NCHP)
        def _():
            R.swait(0)
            R.start(0, j + 2)

        @pl.when(j + 3 < _NCHP)
        def _():
            R.swait(1)
            R.start(1, j + 3)

        @pl.when(t < nhalf)
        def _():
            G.gwait(0)
            G.scat(0, gb + j)
            G.gwait(1)
            G.scat(1, gb + j + 1)
            G.swait(0)
            G.start(0, gb + j + 2)

            @pl.when(j + 3 < _NCHL)
            def _():
                G.swait(1)
                G.start(1, gb + j + 3)

            L.gwait(0)
            L.scat(0, j)
            L.gwait(1)
            L.scat(1, j + 1)
            L.swait(0)
            L.start(0, j + 2)

            @pl.when(j + 3 < _NCHL)
            def _():
                L.swait(1)
                L.start(1, j + 3)

        return 0

    lax.fori_loop(0, _NCHP // 2, body, 0)
    G.gwait(0)
    G.scat(0, gb + _NCHL - 1)
    L.gwait(0)
    L.scat(0, _NCHL - 1)
    R.swait(0)
    R.swait(1)
    G.swait(0)
    G.swait(1)
    L.swait(0)
    L.swait(1)

    plsc.subcore_barrier()
    sl = pl.ds(s * _RPT, _RPT)
    pltpu.sync_copy(accp_s.at[sl], aggp_hbm.at[c, sl])
    pltpu.sync_copy(gacc_s.at[sl], g_hbm.at[c, sl])
    pltpu.sync_copy(accl_s.at[sl], aggl_hbm.at[c, sl])


_mid_call = pl.kernel(
    _mid_body,
    out_type=[jax.ShapeDtypeStruct((_NC, _NPAD, _HD), _f32),
              jax.ShapeDtypeStruct((_NC, _NPAD), _f32),
              jax.ShapeDtypeStruct((_NC, _NPAD, 16), _f32)],
    mesh=_sc_mesh,
    compiler_params=_sc_params,
    scratch_types=[
        pltpu.VMEM((_NCHP, _CH), jnp.int32),
        pltpu.VMEM((_NCHP, _CH), jnp.int32),
        pltpu.VMEM((_NCHL, _CH), jnp.int32),
        pltpu.VMEM((_NCHL, _CH), jnp.int32),
        pltpu.VMEM((_CH, _HD), _f32),
        pltpu.VMEM((_CH, _HD), _f32),
        pltpu.VMEM((_CH,), _f32),
        pltpu.VMEM((_CH,), _f32),
        pltpu.VMEM((_CH, 16), _f32),
        pltpu.VMEM((_CH, 16), _f32),
        pltpu.VMEM_SHARED((_NPAD, _HD), _f32),
        pltpu.VMEM_SHARED((_NPAD,), _f32),
        pltpu.VMEM_SHARED((_NPAD, 16), _f32),
    ] + [pltpu.SemaphoreType.DMA] * 12)


def _l2_body2(y2_hbm, srcl_hbm, dstl_hbm, agg_hbm,
              sidx, didx, gb0, gb1, acc_s, gs0, gs1, ss0, ss1):
    c = lax.axis_index("c")
    s = lax.axis_index("s")
    _zero_acc2d(gb0, acc_s, s, _HD)
    plsc.subcore_barrier()
    pltpu.sync_copy(srcl_hbm.at[s], sidx)
    pltpu.sync_copy(dstl_hbm.at[s], didx)
    st = _Stream(y2_hbm.at[c], sidx, didx, acc_s, gb0, gb1,
                 gs0, gs1, ss0, ss1)
    _run_stream(st, _NCHP, 0)
    plsc.subcore_barrier()
    sl = pl.ds(s * _RPT, _RPT)
    pltpu.sync_copy(acc_s.at[sl], agg_hbm.at[c, sl])


_l2_call = pl.kernel(
    _l2_body2,
    out_type=[jax.ShapeDtypeStruct((_NC, _NPAD, _HD), _f32)],
    mesh=_sc_mesh,
    compiler_params=_sc_params,
    scratch_types=[
        pltpu.VMEM((_NCHP, _CH), jnp.int32),
        pltpu.VMEM((_NCHP, _CH), jnp.int32),
        pltpu.VMEM((_CH, _HD), _f32),
        pltpu.VMEM((_CH, _HD), _f32),
        pltpu.VMEM_SHARED((_NPAD, _HD), _f32),
        pltpu.SemaphoreType.DMA,
        pltpu.SemaphoreType.DMA,
        pltpu.SemaphoreType.DMA,
        pltpu.SemaphoreType.DMA,
    ])


def _prep_body(degp_ref, degl_ref, xp_ref, xl_ref,
               yp_ref, yl_ref, invp_ref, invl_ref):
    invp = lax.rsqrt(degp_ref[...])
    invl = lax.rsqrt(degl_ref[...])
    invp_ref[...] = invp
    invl_ref[...] = invl
    yp = xp_ref[...] * invp
    yp_ref[0, :, :] = yp[:, :_HD]
    yp_ref[1, :, :] = yp[:, _HD:]
    xl16 = jnp.concatenate(
        [xl_ref[...], jnp.zeros((_BM, 1), _f32)], axis=1)
    yl_ref[...] = xl16 * invl


def _prep_call(degp, degl, x_p, x_l):
    return pl.pallas_call(
        _prep_body,
        grid=(_GRID,),
        in_specs=[pl.BlockSpec((_BM, 1), lambda i: (i, 0)),
                  pl.BlockSpec((_BM, 1), lambda i: (i, 0)),
                  pl.BlockSpec((_BM, 128), lambda i: (i, 0)),
                  pl.BlockSpec((_BM, 15), lambda i: (i, 0))],
        out_specs=[pl.BlockSpec((_NC, _BM, _HD), lambda i: (0, i, 0)),
                   pl.BlockSpec((_BM, 16), lambda i: (i, 0)),
                   pl.BlockSpec((_BM, 1), lambda i: (i, 0)),
                   pl.BlockSpec((_BM, 1), lambda i: (i, 0))],
        out_shape=[jax.ShapeDtypeStruct((_NC, _NPAD, _HD), _f32),
                   jax.ShapeDtypeStruct((_NPAD, 16), _f32),
                   jax.ShapeDtypeStruct((_NPAD, 1), _f32),
                   jax.ShapeDtypeStruct((_NPAD, 1), _f32)],
    )(degp, degl, x_p, x_l)


def _mid_tc_body(ap_ref, yp_ref, invp_ref, g_ref,
                 al_ref, yl_ref, invl_ref,
                 wp1_ref, bp1_ref, wp2_ref, bp2_ref, wl1_ref, bl1_ref,
                 y2_ref, zp_ref, sacc_ref):
    i = pl.program_id(0)
    invp = invp_ref[...]
    aggp = invp * jnp.concatenate(
        [ap_ref[0] + yp_ref[0], ap_ref[1] + yp_ref[1]], axis=1)
    h = jnp.maximum(
        jnp.dot(aggp, wp1_ref[...], preferred_element_type=_f32)
        + bp1_ref[...], 0.0)
    rowid = lax.broadcasted_iota(jnp.int32, (_BM, 1), 0) + i * _BM
    cvec = jnp.where(rowid < _N, invp * (g_ref[...] + invp), 0.0)

    @pl.when(i == 0)
    def _():
        sacc_ref[...] = jnp.zeros_like(sacc_ref)

    sacc_ref[...] += jnp.sum(cvec * h, axis=0, keepdims=True)

    invl = invl_ref[...]
    aggl = invl * (al_ref[0] + al_ref[1] + yl_ref[...])
    hl = jnp.maximum(
        jnp.dot(aggl, wl1_ref[...], preferred_element_type=_f32)
        + bl1_ref[...], 0.0)
    y2 = invl * hl
    y2_ref[0, :, :] = y2[:, :_HD]
    y2_ref[1, :, :] = y2[:, _HD:]

    @pl.when(i == _GRID - 1)
    def _():
        zp_ref[...] = jnp.dot(sacc_ref[...] / _N, wp2_ref[...],
                              preferred_element_type=_f32) + bp2_ref[...]


def _mid_tc_call(aggp, yp3, invp, gmat, aggl, yl, invl,
                 Wp1, bp1, Wp2, bp2, Wl1p, bl1):
    blk = lambda r, c: pl.BlockSpec((r, c), lambda i: (i, 0))
    blk3 = lambda c: pl.BlockSpec((_NC, _BM, c), lambda i: (0, i, 0))
    cst = lambda r, c: pl.BlockSpec((r, c), lambda i: (0, 0))
    return pl.pallas_call(
        _mid_tc_body,
        grid=(_GRID,),
        in_specs=[blk3(_HD), blk3(_HD), blk(_BM, 1), blk(_BM, 1),
                  blk3(16), blk(_BM, 16), blk(_BM, 1),
                  cst(128, 128), cst(1, 128), cst(128, 128), cst(1, 128),
                  cst(16, 128), cst(1, 128)],
        out_specs=[blk3(_HD), cst(1, 128)],
        out_shape=[jax.ShapeDtypeStruct((_NC, _NPAD, _HD), _f32),
                   jax.ShapeDtypeStruct((1, 128), _f32)],
        scratch_shapes=[pltpu.VMEM((1, 128), _f32)],
    )(aggp, yp3, invp, gmat, aggl, yl, invl,
      Wp1, bp1[None, :], Wp2, bp2[None, :], Wl1p, bl1[None, :])


def _fin_body(bfs_ref, q_ref, y2_ref, invl_ref, lab_ref,
              labf_ref, wl2_ref, bl2_ref, wf_ref, bf_ref,
              wd1_ref, bd1_ref, wd2_ref, bd2_ref,
              zv_ref, lp_ref, zlsum_ref, zvsum_ref, labA_ref, labB_ref):
    i = pl.program_id(0)
    rowid = lax.broadcasted_iota(jnp.int32, (_BM, 1), 0) + i * _BM

    @pl.when(i == 0)
    def _():
        lp_ref[...] = jnp.zeros_like(lp_ref)
        zlsum_ref[...] = jnp.zeros_like(zlsum_ref)
        zvsum_ref[...] = jnp.zeros_like(zvsum_ref)
        labA_ref[...] = jnp.zeros_like(labA_ref)
        labB_ref[...] = jnp.zeros_like(labB_ref)

    invl = invl_ref[...]
    aggl2 = invl * jnp.concatenate(
        [q_ref[0] + y2_ref[0], q_ref[1] + y2_ref[1]], axis=1)
    zl = jnp.dot(aggl2, wl2_ref[...], preferred_element_type=_f32) \
        + bl2_ref[...]
    logits = jnp.dot(zl, wf_ref[...], preferred_element_type=_f32) \
        + bf_ref[...]
    m = jnp.max(logits, axis=1, keepdims=True)
    e = jnp.exp(logits - m)
    lab = lab_ref[...]
    num = jnp.sum(e * lab, axis=1, keepdims=True)
    den = jnp.sum(e, axis=1, keepdims=True)
    lig_mask = rowid < _N
    inner = jnp.where(lig_mask, num / den, 1.0)
    lp_ref[...] += jnp.sum(jnp.log(inner), axis=0, keepdims=True)
    zlsum_ref[...] += jnp.sum(jnp.where(lig_mask, zl, 0.0), axis=0,
                              keepdims=True)
    labA_ref[...] += jnp.sum(jnp.where(lig_mask, lab, 0.0), axis=0,
                             keepdims=True)
    labB_ref[...] += jnp.sum(jnp.where(rowid < _ND, lab, 0.0), axis=0,
                             keepdims=True)

    src0 = bfs_ref[0, 0]
    dst0 = bfs_ref[1, 0]
    same = src0 == dst0
    isq = 0.70710678118654752
    x_s0 = labf_ref[pl.ds(src0, 1), :]
    x_d0 = labf_ref[pl.ds(dst0, 1), :]
    agg_d = jnp.where(same, x_d0, isq * x_s0 + 0.5 * x_d0)
    h_spec = jnp.maximum(
        jnp.dot(agg_d, wd1_ref[...], preferred_element_type=_f32)
        + bd1_ref[...], 0.0)
    h_src0 = jnp.maximum(
        jnp.dot(x_s0, wd1_ref[...], preferred_element_type=_f32)
        + bd1_ref[...], 0.0)
    agg2_d = jnp.where(same, h_spec, isq * h_src0 + 0.5 * h_spec)
    z_spec = jnp.dot(agg2_d, wd2_ref[...], preferred_element_type=_f32) \
        + bd2_ref[...]
    h = jnp.maximum(
        jnp.dot(lab, wd1_ref[...], preferred_element_type=_f32)
        + bd1_ref[...], 0.0)
    h = jnp.where(rowid == dst0, h_spec, h)
    zv = jnp.dot(h, wd2_ref[...], preferred_element_type=_f32) \
        + bd2_ref[...]
    zv = jnp.where(rowid == dst0, z_spec, zv)
    zv_ref[...] = zv
    zvsum_ref[...] += jnp.sum(jnp.where(rowid < _ND, zv, 0.0), axis=0,
                              keepdims=True)


def _fin_call(bfs, aggl2, y23, invl, lab16, Wl2, bl2, Wf16, bf16,
              Wd1p, bd1, Wd2, bd2):
    blk = lambda r, c: pl.BlockSpec((r, c), lambda i: (i, 0))
    blk3 = lambda c: pl.BlockSpec((_NC, _BM, c), lambda i: (0, i, 0))
    cst = lambda r, c: pl.BlockSpec((r, c), lambda i: (0, 0))
    return pl.pallas_call(
        _fin_body,
        grid=(_GRID,),
        in_specs=[pl.BlockSpec(memory_space=pltpu.SMEM),
                  blk3(_HD), blk3(_HD), blk(_BM, 1),
                  blk(_BM, 16),
                  cst(_NPAD, 16),
                  cst(128, 128), cst(1, 128), cst(128, 16), cst(1, 16),
                  cst(16, 128), cst(1, 128), cst(128, 128), cst(1, 128)],
        out_specs=[blk(_BM, 128), cst(1, 1), cst(1, 128), cst(1, 128),
                   cst(1, 16), cst(1, 16)],
        out_shape=[jax.ShapeDtypeStruct((_ND, 128), _f32),
                   jax.ShapeDtypeStruct((1, 1), _f32),
                   jax.ShapeDtypeStruct((1, 128), _f32),
                   jax.ShapeDtypeStruct((1, 128), _f32),
                   jax.ShapeDtypeStruct((1, 16), _f32),
                   jax.ShapeDtypeStruct((1, 16), _f32)],
    )(bfs, aggl2, y23, invl, lab16, lab16, Wl2, bl2[None, :], Wf16,
      bf16[None, :], Wd1p, bd1[None, :], Wd2, bd2[None, :])


def kernel(x_p, edge_index_p, x_l, edge_index_l, bfs_init, Wp1, bp1, Wp2,
           bp2, Wl1, bl1, Wl2, bl2, Wd1, bd1, Wd2, bd2, Wf, bf):
    srcp16 = edge_index_p[0].reshape(_NS, _NCHP, _CH)
    dstp16 = edge_index_p[1].reshape(_NS, _NCHP, _CH)
    srcl16 = edge_index_l[0].reshape(_NS, _NCHP, _CH)
    dstl16 = edge_index_l[1].reshape(_NS, _NCHP, _CH)
    srcl32 = edge_index_l[0].reshape(_NC, _NS, _NCHL, _CH)
    dstl32 = edge_index_l[1].reshape(_NC, _NS, _NCHL, _CH)
    dstp32 = edge_index_p[1].reshape(_NC, _NS, _NCHL, _CH)

    dp, dl = _deg_call(dstp32, dstl32)
    degp_col = (dp[0] + dp[1] + 1.0)[:, None]
    degl_col = (dl[0] + dl[1] + 1.0)[:, None]

    yp3, yl, invp, invl = _prep_call(degp_col, degl_col, x_p, x_l)

    aggp, gmat, aggl = _mid_call(yp3, srcp16, dstp16,
                                 invp.reshape(_NPAD), yl, srcl32, dstl32)
    gcol = (gmat[0] + gmat[1])[:, None]

    Wl1p = jnp.pad(Wl1, ((0, 1), (0, 0)))
    y23, zp = _mid_tc_call(aggp, yp3, invp, gcol, aggl, yl, invl,
                           Wp1, bp1, Wp2, bp2, Wl1p, bl1)
    z_pocket = zp[0]

    (aggl2,) = _l2_call(y23, srcl16, dstl16)

    lab_v = x_l[:, 4:]
    stop_row = jnp.zeros((1, 16), _f32).at[0, _NATOM - 1].set(1.0)
    lab16 = jnp.concatenate([
        jnp.pad(lab_v, ((0, 0), (0, 5))), stop_row,
        jnp.zeros((_NPAD - _ND, 16), _f32)], axis=0)
    Wf16 = jnp.pad(Wf, ((0, 0), (0, 5)))
    bf16 = jnp.concatenate([bf[:_NATOM - 1], bf[_NATOM - 1:] - 1e9,
                            jnp.full((5,), -1e9, _f32)])
    Wd1p = jnp.pad(Wd1, ((0, 5), (0, 0)))
    z_v, lp, zlsum, zvsum, labA, labB = _fin_call(
        bfs_init, aggl2, y23, invl, lab16,
        Wl2, bl2, Wf16, bf16, Wd1p, bd1, Wd2, bd2)

    log_prob = lp[0, 0]
    H_init = jnp.concatenate([zlsum[0] / _N, labA[0, :_NATOM] / _N])
    H_t = jnp.concatenate([zvsum[0] / _ND, labB[0, :_NATOM] / _ND])
    return (log_prob, z_pocket, z_v, H_init, H_t)


# speedup vs baseline: 35.9043x; 1.1177x over previous
import functools

import jax
import jax.numpy as jnp
from jax import lax
from jax.experimental import pallas as pl
from jax.experimental.pallas import tpu as pltpu
from jax.experimental.pallas import tpu_sc as plsc

_N = 10000
_E = 320000
_NPAD = 10240
_NATOM = 11
_ND = 10001

_NC, _NS, _LANES = 2, 16, 16
_CH = 100
_NCHL = _E // (_NC * _NS * _CH)
_NCHP = _E // (_NS * _CH)
_RPT = _NPAD // _NS
_HD = 64

_BM = 512
_GRID = _NPAD // _BM

_f32 = jnp.float32

_sc_mesh = plsc.VectorSubcoreMesh(
    core_axis_name="c", subcore_axis_name="s",
    num_cores=_NC, num_subcores=_NS)
_sc_params = pltpu.CompilerParams(use_tc_tiling_on_sc=False)


def _fill_1d(buf, n, value):
    v = jnp.full((_LANES,), value, _f32)

    def body(i, _):
        buf[pl.ds(i * _LANES, _LANES)] = v
        return 0

    lax.fori_loop(0, n // _LANES, body, 0)


def _fill_2d(buf, rows, cols, value):
    v = jnp.full((_LANES,), value, _f32)
    nseg = cols // _LANES

    def body(i, _):
        for k in range(nseg):
            buf[i, pl.ds(k * _LANES, _LANES)] = v
        return 0

    lax.fori_loop(0, rows, body, 0)


class _Stream:
    def __init__(self, y_hbm, gidx, sidx2, acc_s, b0, b1, gs0, gs1):
        self.y_hbm = y_hbm
        self.gidx = gidx
        self.sidx = sidx2
        self.acc = acc_s
        self.b = (b0, b1)
        self.gs = (gs0, gs1)

    def start(self, k, j):
        pltpu.async_copy(self.y_hbm.at[self.gidx.at[j]], self.b[k],
                         self.gs[k])

    def gwait(self, k):
        pltpu.make_async_copy(self.y_hbm.at[pl.ds(0, _CH)], self.b[k],
                              self.gs[k]).wait()

    def scat(self, k, j):
        pltpu.sync_copy(self.b[k], self.acc.at[self.sidx.at[j]],
                        add=True)


def _run_stream(st, nch, base):
    st.start(0, base)
    st.start(1, base + 1)

    def body(t, _):
        j = base + 2 * t
        st.gwait(0)
        st.scat(0, j)

        @pl.when(j + 2 < base + nch)
        def _():
            st.start(0, j + 2)

        st.gwait(1)
        st.scat(1, j + 1)

        @pl.when(j + 3 < base + nch)
        def _():
            st.start(1, j + 3)

        return 0

    lax.fori_loop(0, nch // 2, body, 0)


def _deg_body(dstp_hbm, dstl_hbm, outp_hbm, outl_hbm,
              idx_v, ones_v, zed_v, histp_s, histl_s, sem):
    c = lax.axis_index("c")
    s = lax.axis_index("s")
    _fill_1d(zed_v, _RPT, 0.0)
    _fill_1d(ones_v, 112, 1.0)
    pltpu.sync_copy(zed_v, histp_s.at[pl.ds(s * _RPT, _RPT)])
    pltpu.sync_copy(zed_v, histl_s.at[pl.ds(s * _RPT, _RPT)])
    plsc.subcore_barrier()

    ones_sl = ones_v.at[pl.ds(0, _CH)]

    def scatter_ones(hist_s):
        def wait_one():
            pltpu.make_async_copy(ones_sl, hist_s.at[idx_v.at[0]],
                                  sem).wait()

        def body(j, _):
            @pl.when(j >= 5)
            def _():
                wait_one()

            pltpu.async_copy(ones_sl, hist_s.at[idx_v.at[j]], sem,
                             add=True)
            return 0

        lax.fori_loop(0, _NCHL, body, 0)
        for _k in range(5):
            wait_one()

    pltpu.sync_copy(dstp_hbm.at[c, s], idx_v)
    scatter_ones(histp_s)
    pltpu.sync_copy(dstl_hbm.at[c, s], idx_v)
    scatter_ones(histl_s)
    plsc.subcore_barrier()
    sl = pl.ds(s * _RPT, _RPT)
    pltpu.sync_copy(histp_s.at[sl], outp_hbm.at[c, sl])
    pltpu.sync_copy(histl_s.at[sl], outl_hbm.at[c, sl])


_deg_call = pl.kernel(
    _deg_body,
    out_type=[jax.ShapeDtypeStruct((_NC, _NPAD), _f32),
              jax.ShapeDtypeStruct((_NC, _NPAD), _f32)],
    mesh=_sc_mesh,
    compiler_params=_sc_params,
    scratch_types=[
        pltpu.VMEM((_NCHL, _CH), jnp.int32),
        pltpu.VMEM((112,), _f32),
        pltpu.VMEM((_RPT,), _f32),
        pltpu.VMEM_SHARED((_NPAD,), _f32),
        pltpu.VMEM_SHARED((_NPAD,), _f32),
        pltpu.SemaphoreType.DMA,
    ])


def _zero_acc2d(gbuf, acc_s, s, cols):
    _fill_2d(gbuf, _CH, cols, 0.0)
    zsl = gbuf.at[pl.ds(0, 80)]
    for k in range(_RPT // 80):
        pltpu.sync_copy(zsl, acc_s.at[pl.ds(s * _RPT + k * 80, 80)])


def _mid_body(yp_hbm, srcp_hbm, dstp_hbm, inv_hbm, yl_hbm, srcl_hbm,
              dstl_hbm, aggp_hbm, g_hbm, aggl_hbm,
              sidx, didx, lsidx, ldidx, rb0, rb1, vb0, vb1, lb0, lb1,
              accp_s, gacc_s, accl_s,
              rgs0, rgs1, ggs0, ggs1, lgs0, lgs1):
    c = lax.axis_index("c")
    s = lax.axis_index("s")
    _zero_acc2d(rb0, accp_s, s, _HD)
    _zero_acc2d(lb0, accl_s, s, 16)
    _fill_1d(vb0, _CH, 0.0)
    zvs = vb0.at[pl.ds(0, 80)]
    for k in range(_RPT // 80):
        pltpu.sync_copy(zvs, gacc_s.at[pl.ds(s * _RPT + k * 80, 80)])
    plsc.subcore_barrier()

    pltpu.sync_copy(srcp_hbm.at[s], sidx)
    pltpu.sync_copy(dstp_hbm.at[s], didx)
    pltpu.sync_copy(srcl_hbm.at[c, s], lsidx)
    pltpu.sync_copy(dstl_hbm.at[c, s], ldidx)

    R = _Stream(yp_hbm.at[c], sidx, didx, accp_s, rb0, rb1,
                rgs0, rgs1)
    G = _Stream(inv_hbm, didx, sidx, gacc_s, vb0, vb1,
                ggs0, ggs1)
    L = _Stream(yl_hbm, lsidx, ldidx, accl_s, lb0, lb1,
                lgs0, lgs1)

    gb = c * _NCHL
    R.start(0, 0)
    R.start(1, 1)
    G.start(0, gb)
    G.start(1, gb + 1)
    L.start(0, 0)
    L.start(1, 1)
    nhalf = _NCHL // 2

    def body(t, _):
        j = 2 * t
        R.gwait(0)
        R.scat(0, j)

        @pl.when(j + 2 < _NCHP)
        def _():
            R.start(0, j + 2)

        R.gwait(1)
        R.scat(1, j + 1)

        @pl.when(j + 3 < _NCHP)
        def _():
            R.start(1, j + 3)

        @pl.when(t < nhalf)
        def _():
            G.gwait(0)
            G.scat(0, gb + j)

            @pl.when(j + 2 < _NCHL)
            def _():
                G.start(0, gb + j + 2)

            G.gwait(1)
            G.scat(1, gb + j + 1)

            @pl.when(j + 3 < _NCHL)
            def _():
                G.start(1, gb + j + 3)

            L.gwait(0)
            L.scat(0, j)

            @pl.when(j + 2 < _NCHL)
            def _():
                L.start(0, j + 2)

            L.gwait(1)
            L.scat(1, j + 1)

            @pl.when(j + 3 < _NCHL)
            def _():
                L.start(1, j + 3)

        return 0

    lax.fori_loop(0, _NCHP // 2, body, 0)

    plsc.subcore_barrier()
    sl = pl.ds(s * _RPT, _RPT)
    pltpu.sync_copy(accp_s.at[sl], aggp_hbm.at[c, sl])
    pltpu.sync_copy(gacc_s.at[sl], g_hbm.at[c, sl])
    pltpu.sync_copy(accl_s.at[sl], aggl_hbm.at[c, sl])


_mid_call = pl.kernel(
    _mid_body,
    out_type=[jax.ShapeDtypeStruct((_NC, _NPAD, _HD), _f32),
              jax.ShapeDtypeStruct((_NC, _NPAD), _f32),
              jax.ShapeDtypeStruct((_NC, _NPAD, 16), _f32)],
    mesh=_sc_mesh,
    compiler_params=_sc_params,
    scratch_types=[
        pltpu.VMEM((_NCHP, _CH), jnp.int32),
        pltpu.VMEM((_NCHP, _CH), jnp.int32),
        pltpu.VMEM((_NCHL, _CH), jnp.int32),
        pltpu.VMEM((_NCHL, _CH), jnp.int32),
        pltpu.VMEM((_CH, _HD), _f32),
        pltpu.VMEM((_CH, _HD), _f32),
        pltpu.VMEM((_CH,), _f32),
        pltpu.VMEM((_CH,), _f32),
        pltpu.VMEM((_CH, 16), _f32),
        pltpu.VMEM((_CH, 16), _f32),
        pltpu.VMEM_SHARED((_NPAD, _HD), _f32),
        pltpu.VMEM_SHARED((_NPAD,), _f32),
        pltpu.VMEM_SHARED((_NPAD, 16), _f32),
    ] + [pltpu.SemaphoreType.DMA] * 6)


def _l2_body2(y2_hbm, srcl_hbm, dstl_hbm, agg_hbm,
              sidx, didx, gb0, gb1, acc_s, gs0, gs1):
    c = lax.axis_index("c")
    s = lax.axis_index("s")
    _zero_acc2d(gb0, acc_s, s, _HD)
    plsc.subcore_barrier()
    pltpu.sync_copy(srcl_hbm.at[s], sidx)
    pltpu.sync_copy(dstl_hbm.at[s], didx)
    st = _Stream(y2_hbm.at[c], sidx, didx, acc_s, gb0, gb1,
                 gs0, gs1)
    _run_stream(st, _NCHP, 0)
    plsc.subcore_barrier()
    sl = pl.ds(s * _RPT, _RPT)
    pltpu.sync_copy(acc_s.at[sl], agg_hbm.at[c, sl])


_l2_call = pl.kernel(
    _l2_body2,
    out_type=[jax.ShapeDtypeStruct((_NC, _NPAD, _HD), _f32)],
    mesh=_sc_mesh,
    compiler_params=_sc_params,
    scratch_types=[
        pltpu.VMEM((_NCHP, _CH), jnp.int32),
        pltpu.VMEM((_NCHP, _CH), jnp.int32),
        pltpu.VMEM((_CH, _HD), _f32),
        pltpu.VMEM((_CH, _HD), _f32),
        pltpu.VMEM_SHARED((_NPAD, _HD), _f32),
        pltpu.SemaphoreType.DMA,
        pltpu.SemaphoreType.DMA,
    ])


def _prep_body(degp_ref, degl_ref, xp_ref, xl_ref,
               yp_ref, yl_ref, invp_ref, invl_ref):
    invp = lax.rsqrt(degp_ref[...])
    invl = lax.rsqrt(degl_ref[...])
    invp_ref[...] = invp
    invl_ref[...] = invl
    yp = xp_ref[...] * invp
    yp_ref[0, :, :] = yp[:, :_HD]
    yp_ref[1, :, :] = yp[:, _HD:]
    xl16 = jnp.concatenate(
        [xl_ref[...], jnp.zeros((_BM, 1), _f32)], axis=1)
    yl_ref[...] = xl16 * invl


def _prep_call(degp, degl, x_p, x_l):
    return pl.pallas_call(
        _prep_body,
        grid=(_GRID,),
        in_specs=[pl.BlockSpec((_BM, 1), lambda i: (i, 0)),
                  pl.BlockSpec((_BM, 1), lambda i: (i, 0)),
                  pl.BlockSpec((_BM, 128), lambda i: (i, 0)),
                  pl.BlockSpec((_BM, 15), lambda i: (i, 0))],
        out_specs=[pl.BlockSpec((_NC, _BM, _HD), lambda i: (0, i, 0)),
                   pl.BlockSpec((_BM, 16), lambda i: (i, 0)),
                   pl.BlockSpec((_BM, 1), lambda i: (i, 0)),
                   pl.BlockSpec((_BM, 1), lambda i: (i, 0))],
        out_shape=[jax.ShapeDtypeStruct((_NC, _NPAD, _HD), _f32),
                   jax.ShapeDtypeStruct((_NPAD, 16), _f32),
                   jax.ShapeDtypeStruct((_NPAD, 1), _f32),
                   jax.ShapeDtypeStruct((_NPAD, 1), _f32)],
    )(degp, degl, x_p, x_l)


def _mid_tc_body(ap_ref, yp_ref, invp_ref, g_ref,
                 al_ref, yl_ref, invl_ref,
                 wp1_ref, bp1_ref, wp2_ref, bp2_ref, wl1_ref, bl1_ref,
                 y2_ref, zp_ref, sacc_ref):
    i = pl.program_id(0)
    invp = invp_ref[...]
    aggp = invp * jnp.concatenate(
        [ap_ref[0] + yp_ref[0], ap_ref[1] + yp_ref[1]], axis=1)
    h = jnp.maximum(
        jnp.dot(aggp, wp1_ref[...], preferred_element_type=_f32)
        + bp1_ref[...], 0.0)
    rowid = lax.broadcasted_iota(jnp.int32, (_BM, 1), 0) + i * _BM
    cvec = jnp.where(rowid < _N, invp * (g_ref[...] + invp), 0.0)

    @pl.when(i == 0)
    def _():
        sacc_ref[...] = jnp.zeros_like(sacc_ref)

    sacc_ref[...] += jnp.sum(cvec * h, axis=0, keepdims=True)

    invl = invl_ref[...]
    aggl = invl * (al_ref[0] + al_ref[1] + yl_ref[...])
    hl = jnp.maximum(
        jnp.dot(aggl, wl1_ref[...], preferred_element_type=_f32)
        + bl1_ref[...], 0.0)
    y2 = invl * hl
    y2_ref[0, :, :] = y2[:, :_HD]
    y2_ref[1, :, :] = y2[:, _HD:]

    @pl.when(i == _GRID - 1)
    def _():
        zp_ref[...] = jnp.dot(sacc_ref[...] / _N, wp2_ref[...],
                              preferred_element_type=_f32) + bp2_ref[...]


def _mid_tc_call(aggp, yp3, invp, gmat, aggl, yl, invl,
                 Wp1, bp1, Wp2, bp2, Wl1p, bl1):
    blk = lambda r, c: pl.BlockSpec((r, c), lambda i: (i, 0))
    blk3 = lambda c: pl.BlockSpec((_NC, _BM, c), lambda i: (0, i, 0))
    cst = lambda r, c: pl.BlockSpec((r, c), lambda i: (0, 0))
    return pl.pallas_call(
        _mid_tc_body,
        grid=(_GRID,),
        in_specs=[blk3(_HD), blk3(_HD), blk(_BM, 1), blk(_BM, 1),
                  blk3(16), blk(_BM, 16), blk(_BM, 1),
                  cst(128, 128), cst(1, 128), cst(128, 128), cst(1, 128),
                  cst(16, 128), cst(1, 128)],
        out_specs=[blk3(_HD), cst(1, 128)],
        out_shape=[jax.ShapeDtypeStruct((_NC, _NPAD, _HD), _f32),
                   jax.ShapeDtypeStruct((1, 128), _f32)],
        scratch_shapes=[pltpu.VMEM((1, 128), _f32)],
    )(aggp, yp3, invp, gmat, aggl, yl, invl,
      Wp1, bp1[None, :], Wp2, bp2[None, :], Wl1p, bl1[None, :])


def _fin_body(bfs_ref, q_ref, y2_ref, invl_ref, lab_ref,
              labf_ref, wl2_ref, bl2_ref, wf_ref, bf_ref,
              wd1_ref, bd1_ref, wd2_ref, bd2_ref,
              zv_ref, lp_ref, zlsum_ref, zvsum_ref, labA_ref, labB_ref):
    i = pl.program_id(0)
    rowid = lax.broadcasted_iota(jnp.int32, (_BM, 1), 0) + i * _BM

    @pl.when(i == 0)
    def _():
        lp_ref[...] = jnp.zeros_like(lp_ref)
        zlsum_ref[...] = jnp.zeros_like(zlsum_ref)
        zvsum_ref[...] = jnp.zeros_like(zvsum_ref)
        labA_ref[...] = jnp.zeros_like(labA_ref)
        labB_ref[...] = jnp.zeros_like(labB_ref)

    invl = invl_ref[...]
    aggl2 = invl * jnp.concatenate(
        [q_ref[0] + y2_ref[0], q_ref[1] + y2_ref[1]], axis=1)
    zl = jnp.dot(aggl2, wl2_ref[...], preferred_element_type=_f32) \
        + bl2_ref[...]
    logits = jnp.dot(zl, wf_ref[...], preferred_element_type=_f32) \
        + bf_ref[...]
    m = jnp.max(logits, axis=1, keepdims=True)
    e = jnp.exp(logits - m)
    lab = lab_ref[...]
    num = jnp.sum(e * lab, axis=1, keepdims=True)
    den = jnp.sum(e, axis=1, keepdims=True)
    lig_mask = rowid < _N
    inner = jnp.where(lig_mask, num / den, 1.0)
    lp_ref[...] += jnp.sum(jnp.log(inner), axis=0, keepdims=True)
    zlsum_ref[...] += jnp.sum(jnp.where(lig_mask, zl, 0.0), axis=0,
                              keepdims=True)
    labA_ref[...] += jnp.sum(jnp.where(lig_mask, lab, 0.0), axis=0,
                             keepdims=True)
    labB_ref[...] += jnp.sum(jnp.where(rowid < _ND, lab, 0.0), axis=0,
                             keepdims=True)

    src0 = bfs_ref[0, 0]
    dst0 = bfs_ref[1, 0]
    same = src0 == dst0
    isq = 0.70710678118654752
    x_s0 = labf_ref[pl.ds(src0, 1), :]
    x_d0 = labf_ref[pl.ds(dst0, 1), :]
    agg_d = jnp.where(same, x_d0, isq * x_s0 + 0.5 * x_d0)
    h_spec = jnp.maximum(
        jnp.dot(agg_d, wd1_ref[...], preferred_element_type=_f32)
        + bd1_ref[...], 0.0)
    h_src0 = jnp.maximum(
        jnp.dot(x_s0, wd1_ref[...], preferred_element_type=_f32)
        + bd1_ref[...], 0.0)
    agg2_d = jnp.where(same, h_spec, isq * h_src0 + 0.5 * h_spec)
    z_spec = jnp.dot(agg2_d, wd2_ref[...], preferred_element_type=_f32) \
        + bd2_ref[...]
    h = jnp.maximum(
        jnp.dot(lab, wd1_ref[...], preferred_element_type=_f32)
        + bd1_ref[...], 0.0)
    h = jnp.where(rowid == dst0, h_spec, h)
    zv = jnp.dot(h, wd2_ref[...], preferred_element_type=_f32) \
        + bd2_ref[...]
    zv = jnp.where(rowid == dst0, z_spec, zv)
    zv_ref[...] = zv
    zvsum_ref[...] += jnp.sum(jnp.where(rowid < _ND, zv, 0.0), axis=0,
                              keepdims=True)


def _fin_call(bfs, aggl2, y23, invl, lab16, Wl2, bl2, Wf16, bf16,
              Wd1p, bd1, Wd2, bd2):
    blk = lambda r, c: pl.BlockSpec((r, c), lambda i: (i, 0))
    blk3 = lambda c: pl.BlockSpec((_NC, _BM, c), lambda i: (0, i, 0))
    cst = lambda r, c: pl.BlockSpec((r, c), lambda i: (0, 0))
    return pl.pallas_call(
        _fin_body,
        grid=(_GRID,),
        in_specs=[pl.BlockSpec(memory_space=pltpu.SMEM),
                  blk3(_HD), blk3(_HD), blk(_BM, 1),
                  blk(_BM, 16),
                  cst(_NPAD, 16),
                  cst(128, 128), cst(1, 128), cst(128, 16), cst(1, 16),
                  cst(16, 128), cst(1, 128), cst(128, 128), cst(1, 128)],
        out_specs=[blk(_BM, 128), cst(1, 1), cst(1, 128), cst(1, 128),
                   cst(1, 16), cst(1, 16)],
        out_shape=[jax.ShapeDtypeStruct((_ND, 128), _f32),
                   jax.ShapeDtypeStruct((1, 1), _f32),
                   jax.ShapeDtypeStruct((1, 128), _f32),
                   jax.ShapeDtypeStruct((1, 128), _f32),
                   jax.ShapeDtypeStruct((1, 16), _f32),
                   jax.ShapeDtypeStruct((1, 16), _f32)],
    )(bfs, aggl2, y23, invl, lab16, lab16, Wl2, bl2[None, :], Wf16,
      bf16[None, :], Wd1p, bd1[None, :], Wd2, bd2[None, :])


def kernel(x_p, edge_index_p, x_l, edge_index_l, bfs_init, Wp1, bp1, Wp2,
           bp2, Wl1, bl1, Wl2, bl2, Wd1, bd1, Wd2, bd2, Wf, bf):
    srcp16 = edge_index_p[0].reshape(_NS, _NCHP, _CH)
    dstp16 = edge_index_p[1].reshape(_NS, _NCHP, _CH)
    srcl16 = edge_index_l[0].reshape(_NS, _NCHP, _CH)
    dstl16 = edge_index_l[1].reshape(_NS, _NCHP, _CH)
    srcl32 = edge_index_l[0].reshape(_NC, _NS, _NCHL, _CH)
    dstl32 = edge_index_l[1].reshape(_NC, _NS, _NCHL, _CH)
    dstp32 = edge_index_p[1].reshape(_NC, _NS, _NCHL, _CH)

    dp, dl = _deg_call(dstp32, dstl32)
    degp_col = (dp[0] + dp[1] + 1.0)[:, None]
    degl_col = (dl[0] + dl[1] + 1.0)[:, None]

    yp3, yl, invp, invl = _prep_call(degp_col, degl_col, x_p, x_l)

    aggp, gmat, aggl = _mid_call(yp3, srcp16, dstp16,
                                 invp.reshape(_NPAD), yl, srcl32, dstl32)
    gcol = (gmat[0] + gmat[1])[:, None]

    Wl1p = jnp.pad(Wl1, ((0, 1), (0, 0)))
    y23, zp = _mid_tc_call(aggp, yp3, invp, gcol, aggl, yl, invl,
                           Wp1, bp1, Wp2, bp2, Wl1p, bl1)
    z_pocket = zp[0]

    (aggl2,) = _l2_call(y23, srcl16, dstl16)

    lab_v = x_l[:, 4:]
    stop_row = jnp.zeros((1, 16), _f32).at[0, _NATOM - 1].set(1.0)
    lab16 = jnp.concatenate([
        jnp.pad(lab_v, ((0, 0), (0, 5))), stop_row,
        jnp.zeros((_NPAD - _ND, 16), _f32)], axis=0)
    Wf16 = jnp.pad(Wf, ((0, 0), (0, 5)))
    bf16 = jnp.concatenate([bf[:_NATOM - 1], bf[_NATOM - 1:] - 1e9,
                            jnp.full((5,), -1e9, _f32)])
    Wd1p = jnp.pad(Wd1, ((0, 5), (0, 0)))
    z_v, lp, zlsum, zvsum, labA, labB = _fin_call(
        bfs_init, aggl2, y23, invl, lab16,
        Wl2, bl2, Wf16, bf16, Wd1p, bd1, Wd2, bd2)

    log_prob = lp[0, 0]
    H_init = jnp.concatenate([zlsum[0] / _N, labA[0, :_NATOM] / _N])
    H_t = jnp.concatenate([zvsum[0] / _ND, labB[0, :_NATOM] / _ND])
    return (log_prob, z_pocket, z_v, H_init, H_t)


# speedup vs baseline: 37.6661x; 1.0491x over previous
import functools

import jax
import jax.numpy as jnp
from jax import lax
from jax.experimental import pallas as pl
from jax.experimental.pallas import tpu as pltpu
from jax.experimental.pallas import tpu_sc as plsc

_N = 10000
_E = 320000
_NPAD = 10240
_NATOM = 11
_ND = 10001

_NC, _NS, _LANES = 2, 16, 16
_CH = 100
_NCHL = _E // (_NC * _NS * _CH)
_NCHP = _E // (_NS * _CH)
_RPT = _NPAD // _NS
_HD = 64

_BM = 512
_GRID = _NPAD // _BM

_f32 = jnp.float32

_sc_mesh = plsc.VectorSubcoreMesh(
    core_axis_name="c", subcore_axis_name="s",
    num_cores=_NC, num_subcores=_NS)
_sc_params = pltpu.CompilerParams(use_tc_tiling_on_sc=False)


def _fill_1d(buf, n, value):
    v = jnp.full((_LANES,), value, _f32)

    def body(i, _):
        buf[pl.ds(i * _LANES, _LANES)] = v
        return 0

    lax.fori_loop(0, n // _LANES, body, 0)


def _fill_2d(buf, rows, cols, value):
    v = jnp.full((_LANES,), value, _f32)
    nseg = cols // _LANES

    def body(i, _):
        for k in range(nseg):
            buf[i, pl.ds(k * _LANES, _LANES)] = v
        return 0

    lax.fori_loop(0, rows, body, 0)


class _Stream:
    def __init__(self, y_hbm, gidx, sidx2, acc_s, b0, b1, gs0, gs1):
        self.y_hbm = y_hbm
        self.gidx = gidx
        self.sidx = sidx2
        self.acc = acc_s
        self.b = (b0, b1)
        self.gs = (gs0, gs1)

    def start(self, k, j):
        pltpu.async_copy(self.y_hbm.at[self.gidx.at[j]], self.b[k],
                         self.gs[k])

    def gwait(self, k):
        pltpu.make_async_copy(self.y_hbm.at[pl.ds(0, _CH)], self.b[k],
                              self.gs[k]).wait()

    def scat(self, k, j):
        pltpu.sync_copy(self.b[k], self.acc.at[self.sidx.at[j]],
                        add=True)


def _run_stream(st, nch, base):
    st.start(0, base)
    st.start(1, base + 1)

    def body(t, _):
        j = base + 2 * t
        st.gwait(0)
        st.scat(0, j)

        @pl.when(j + 2 < base + nch)
        def _():
            st.start(0, j + 2)

        st.gwait(1)
        st.scat(1, j + 1)

        @pl.when(j + 3 < base + nch)
        def _():
            st.start(1, j + 3)

        return 0

    lax.fori_loop(0, nch // 2, body, 0)


def _deg_body(ep_hbm, el_hbm, outp_hbm, outl_hbm,
              idx_v, ones_v, zed_v, histp_s, histl_s, sem):
    c = lax.axis_index("c")
    s = lax.axis_index("s")
    _fill_1d(zed_v, _RPT, 0.0)
    _fill_1d(ones_v, 112, 1.0)
    pltpu.sync_copy(zed_v, histp_s.at[pl.ds(s * _RPT, _RPT)])
    pltpu.sync_copy(zed_v, histl_s.at[pl.ds(s * _RPT, _RPT)])
    plsc.subcore_barrier()

    ones_sl = ones_v.at[pl.ds(0, _CH)]

    def scatter_ones(hist_s):
        def wait_one():
            pltpu.make_async_copy(ones_sl, hist_s.at[idx_v.at[0]],
                                  sem).wait()

        def body(j, _):
            @pl.when(j >= 5)
            def _():
                wait_one()

            pltpu.async_copy(ones_sl, hist_s.at[idx_v.at[j]], sem,
                             add=True)
            return 0

        lax.fori_loop(0, _NCHL, body, 0)
        for _k in range(5):
            wait_one()

    pltpu.sync_copy(ep_hbm.at[1, s, pl.ds(c * _NCHL, _NCHL)], idx_v)
    scatter_ones(histp_s)
    pltpu.sync_copy(el_hbm.at[1, s, pl.ds(c * _NCHL, _NCHL)], idx_v)
    scatter_ones(histl_s)
    plsc.subcore_barrier()
    sl = pl.ds(s * _RPT, _RPT)
    pltpu.sync_copy(histp_s.at[sl], outp_hbm.at[c, sl])
    pltpu.sync_copy(histl_s.at[sl], outl_hbm.at[c, sl])


_deg_call = pl.kernel(
    _deg_body,
    out_type=[jax.ShapeDtypeStruct((_NC, _NPAD), _f32),
              jax.ShapeDtypeStruct((_NC, _NPAD), _f32)],
    mesh=_sc_mesh,
    compiler_params=_sc_params,
    scratch_types=[
        pltpu.VMEM((_NCHL, _CH), jnp.int32),
        pltpu.VMEM((112,), _f32),
        pltpu.VMEM((_RPT,), _f32),
        pltpu.VMEM_SHARED((_NPAD,), _f32),
        pltpu.VMEM_SHARED((_NPAD,), _f32),
        pltpu.SemaphoreType.DMA,
    ])


def _zero_acc2d(gbuf, acc_s, s, cols):
    _fill_2d(gbuf, _CH, cols, 0.0)
    zsl = gbuf.at[pl.ds(0, 80)]
    for k in range(_RPT // 80):
        pltpu.sync_copy(zsl, acc_s.at[pl.ds(s * _RPT + k * 80, 80)])


def _mid_body(yp_hbm, ep_hbm, inv_hbm, yl_hbm, el_hbm,
              aggp_hbm, g_hbm, aggl_hbm,
              sidx, didx, lsidx, ldidx, rb0, rb1, vb0, vb1, lb0, lb1,
              accp_s, gacc_s, accl_s,
              rgs0, rgs1, ggs0, ggs1, lgs0, lgs1):
    c = lax.axis_index("c")
    s = lax.axis_index("s")
    _zero_acc2d(rb0, accp_s, s, _HD)
    _zero_acc2d(lb0, accl_s, s, 16)
    _fill_1d(vb0, _CH, 0.0)
    zvs = vb0.at[pl.ds(0, 80)]
    for k in range(_RPT // 80):
        pltpu.sync_copy(zvs, gacc_s.at[pl.ds(s * _RPT + k * 80, 80)])
    plsc.subcore_barrier()

    pltpu.sync_copy(ep_hbm.at[0, s], sidx)
    pltpu.sync_copy(ep_hbm.at[1, s], didx)
    pltpu.sync_copy(el_hbm.at[0, s, pl.ds(c * _NCHL, _NCHL)], lsidx)
    pltpu.sync_copy(el_hbm.at[1, s, pl.ds(c * _NCHL, _NCHL)], ldidx)

    R = _Stream(yp_hbm.at[c], sidx, didx, accp_s, rb0, rb1, rgs0, rgs1)
    G = _Stream(inv_hbm, didx, sidx, gacc_s, vb0, vb1, ggs0, ggs1)
    L = _Stream(yl_hbm, lsidx, ldidx, accl_s, lb0, lb1, lgs0, lgs1)

    gb = c * _NCHL
    R.start(0, 0)
    R.start(1, 1)
    G.start(0, gb)
    G.start(1, gb + 1)
    L.start(0, 0)
    L.start(1, 1)
    nhalf = _NCHL // 2

    def body(t, _):
        j = 2 * t
        R.gwait(0)
        R.scat(0, j)

        @pl.when(j + 2 < _NCHP)
        def _():
            R.start(0, j + 2)

        R.gwait(1)
        R.scat(1, j + 1)

        @pl.when(j + 3 < _NCHP)
        def _():
            R.start(1, j + 3)

        @pl.when(t < nhalf)
        def _():
            G.gwait(0)
            G.scat(0, gb + j)

            @pl.when(j + 2 < _NCHL)
            def _():
                G.start(0, gb + j + 2)

            G.gwait(1)
            G.scat(1, gb + j + 1)

            @pl.when(j + 3 < _NCHL)
            def _():
                G.start(1, gb + j + 3)

            L.gwait(0)
            L.scat(0, j)

            @pl.when(j + 2 < _NCHL)
            def _():
                L.start(0, j + 2)

            L.gwait(1)
            L.scat(1, j + 1)

            @pl.when(j + 3 < _NCHL)
            def _():
                L.start(1, j + 3)

        return 0

    lax.fori_loop(0, _NCHP // 2, body, 0)

    plsc.subcore_barrier()
    sl = pl.ds(s * _RPT, _RPT)
    pltpu.sync_copy(accp_s.at[sl], aggp_hbm.at[c, sl])
    pltpu.sync_copy(gacc_s.at[sl], g_hbm.at[c, sl])
    pltpu.sync_copy(accl_s.at[sl], aggl_hbm.at[c, sl])


_mid_call = pl.kernel(
    _mid_body,
    out_type=[jax.ShapeDtypeStruct((_NC, _NPAD, _HD), _f32),
              jax.ShapeDtypeStruct((_NC, _NPAD), _f32),
              jax.ShapeDtypeStruct((_NC, _NPAD, 16), _f32)],
    mesh=_sc_mesh,
    compiler_params=_sc_params,
    scratch_types=[
        pltpu.VMEM((_NCHP, _CH), jnp.int32),
        pltpu.VMEM((_NCHP, _CH), jnp.int32),
        pltpu.VMEM((_NCHL, _CH), jnp.int32),
        pltpu.VMEM((_NCHL, _CH), jnp.int32),
        pltpu.VMEM((_CH, _HD), _f32),
        pltpu.VMEM((_CH, _HD), _f32),
        pltpu.VMEM((_CH,), _f32),
        pltpu.VMEM((_CH,), _f32),
        pltpu.VMEM((_CH, 16), _f32),
        pltpu.VMEM((_CH, 16), _f32),
        pltpu.VMEM_SHARED((_NPAD, _HD), _f32),
        pltpu.VMEM_SHARED((_NPAD,), _f32),
        pltpu.VMEM_SHARED((_NPAD, 16), _f32),
    ] + [pltpu.SemaphoreType.DMA] * 6)


def _l2_body(y2_hbm, el_hbm, agg_hbm,
             sidx, didx, gb0, gb1, acc_s, gs0, gs1):
    c = lax.axis_index("c")
    s = lax.axis_index("s")
    _zero_acc2d(gb0, acc_s, s, _HD)
    plsc.subcore_barrier()
    pltpu.sync_copy(el_hbm.at[0, s], sidx)
    pltpu.sync_copy(el_hbm.at[1, s], didx)
    st = _Stream(y2_hbm.at[c], sidx, didx, acc_s, gb0, gb1, gs0, gs1)
    _run_stream(st, _NCHP, 0)
    plsc.subcore_barrier()
    sl = pl.ds(s * _RPT, _RPT)
    pltpu.sync_copy(acc_s.at[sl], agg_hbm.at[c, sl])


_l2_call = pl.kernel(
    _l2_body,
    out_type=[jax.ShapeDtypeStruct((_NC, _NPAD, _HD), _f32)],
    mesh=_sc_mesh,
    compiler_params=_sc_params,
    scratch_types=[
        pltpu.VMEM((_NCHP, _CH), jnp.int32),
        pltpu.VMEM((_NCHP, _CH), jnp.int32),
        pltpu.VMEM((_CH, _HD), _f32),
        pltpu.VMEM((_CH, _HD), _f32),
        pltpu.VMEM_SHARED((_NPAD, _HD), _f32),
        pltpu.SemaphoreType.DMA,
        pltpu.SemaphoreType.DMA,
    ])


def _prep_body(degp_ref, degl_ref, xp_ref, xl_ref,
               yp_ref, yl_ref, invp_ref, invl_ref):
    invp = lax.rsqrt(degp_ref[...])
    invl = lax.rsqrt(degl_ref[...])
    invp_ref[...] = invp
    invl_ref[...] = invl
    yp = xp_ref[...] * invp
    yp_ref[0, :, :] = yp[:, :_HD]
    yp_ref[1, :, :] = yp[:, _HD:]
    xl16 = jnp.concatenate(
        [xl_ref[...], jnp.zeros((_BM, 1), _f32)], axis=1)
    yl_ref[...] = xl16 * invl


def _prep_call(degp, degl, x_p, x_l):
    blkc = lambda c: pl.BlockSpec((_NC, _BM, c), lambda i: (0, i, 0))
    return pl.pallas_call(
        _prep_body,
        grid=(_GRID,),
        in_specs=[pl.BlockSpec((_BM, 1), lambda i: (i, 0)),
                  pl.BlockSpec((_BM, 1), lambda i: (i, 0)),
                  pl.BlockSpec((_BM, 128), lambda i: (i, 0)),
                  pl.BlockSpec((_BM, 15), lambda i: (i, 0))],
        out_specs=[blkc(_HD),
                   pl.BlockSpec((_BM, 16), lambda i: (i, 0)),
                   pl.BlockSpec((_BM, 1), lambda i: (i, 0)),
                   pl.BlockSpec((_BM, 1), lambda i: (i, 0))],
        out_shape=[jax.ShapeDtypeStruct((_NC, _NPAD, _HD), _f32),
                   jax.ShapeDtypeStruct((_NPAD, 16), _f32),
                   jax.ShapeDtypeStruct((_NPAD, 1), _f32),
                   jax.ShapeDtypeStruct((_NPAD, 1), _f32)],
    )(degp, degl, x_p, x_l)


def _mid_tc_body(ap_ref, yp_ref, invp_ref, g_ref,
                 al_ref, yl_ref, invl_ref,
                 wp1_ref, bp1_ref, wp2_ref, bp2_ref, wl1_ref, bl1_ref,
                 y2_ref, zp_ref, sacc_ref):
    i = pl.program_id(0)
    invp = invp_ref[...]
    aggp = invp * jnp.concatenate(
        [ap_ref[0] + yp_ref[0], ap_ref[1] + yp_ref[1]], axis=1)
    h = jnp.maximum(
        jnp.dot(aggp, wp1_ref[...], preferred_element_type=_f32)
        + bp1_ref[...], 0.0)
    rowid = lax.broadcasted_iota(jnp.int32, (_BM, 1), 0) + i * _BM
    cvec = jnp.where(rowid < _N, invp * (g_ref[...] + invp), 0.0)

    @pl.when(i == 0)
    def _():
        sacc_ref[...] = jnp.zeros_like(sacc_ref)

    sacc_ref[...] += jnp.sum(cvec * h, axis=0, keepdims=True)

    invl = invl_ref[...]
    aggl = invl * (al_ref[0] + al_ref[1] + yl_ref[...])
    wl1p = jnp.concatenate([wl1_ref[...], jnp.zeros((1, 128), _f32)],
                           axis=0)
    hl = jnp.maximum(
        jnp.dot(aggl, wl1p, preferred_element_type=_f32)
        + bl1_ref[...], 0.0)
    y2 = invl * hl
    y2_ref[0, :, :] = y2[:, :_HD]
    y2_ref[1, :, :] = y2[:, _HD:]

    @pl.when(i == _GRID - 1)
    def _():
        zp = jnp.dot(sacc_ref[...] / _N, wp2_ref[...],
                     preferred_element_type=_f32) + bp2_ref[...]
        zp_ref[...] = zp[0]


def _mid_tc_call(aggp, yp3, invp, gmat, aggl, yl, invl,
                 Wp1, bp1, Wp2, bp2, Wl1, bl1):
    blk = lambda r, c: pl.BlockSpec((r, c), lambda i: (i, 0))
    blk3 = lambda c: pl.BlockSpec((_NC, _BM, c), lambda i: (0, i, 0))
    cst = lambda r, c: pl.BlockSpec((r, c), lambda i: (0, 0))
    return pl.pallas_call(
        _mid_tc_body,
        grid=(_GRID,),
        in_specs=[blk3(_HD), blk3(_HD), blk(_BM, 1), blk(_BM, 1),
                  blk3(16), blk(_BM, 16), blk(_BM, 1),
                  cst(128, 128), cst(1, 128), cst(128, 128), cst(1, 128),
                  cst(15, 128), cst(1, 128)],
        out_specs=[blk3(_HD),
                   pl.BlockSpec((128,), lambda i: (0,))],
        out_shape=[jax.ShapeDtypeStruct((_NC, _NPAD, _HD), _f32),
                   jax.ShapeDtypeStruct((128,), _f32)],
        scratch_shapes=[pltpu.VMEM((1, 128), _f32)],
    )(aggp, yp3, invp, gmat, aggl, yl, invl,
      Wp1, bp1[None, :], Wp2, bp2[None, :], Wl1, bl1[None, :])


def _fin_body(bfs_ref, q_ref, y2_ref, invl_ref, xl_ref, xlf_ref,
              wl2_ref, bl2_ref, wf_ref, bf_ref,
              wd1_ref, bd1_ref, wd2_ref, bd2_ref,
              zv_ref, lp_ref, hi_ref, ht_ref,
              zlsum_ref, zvsum_ref, labA_ref, labB_ref):
    i = pl.program_id(0)
    rowid = lax.broadcasted_iota(jnp.int32, (_BM, 1), 0) + i * _BM

    @pl.when(i == 0)
    def _():
        lp_ref[...] = jnp.zeros_like(lp_ref)
        zlsum_ref[...] = jnp.zeros_like(zlsum_ref)
        zvsum_ref[...] = jnp.zeros_like(zvsum_ref)
        labA_ref[...] = jnp.zeros_like(labA_ref)
        labB_ref[...] = jnp.zeros_like(labB_ref)

    stop_row = jnp.where(
        lax.broadcasted_iota(jnp.int32, (1, _NATOM), 1) == _NATOM - 1,
        1.0, 0.0)
    lab = jnp.where(rowid < _N, xl_ref[...][:, 4:], 0.0)
    lab = jnp.where(rowid == _N, stop_row, lab)

    invl = invl_ref[...]
    aggl2 = invl * jnp.concatenate(
        [q_ref[0] + y2_ref[0], q_ref[1] + y2_ref[1]], axis=1)
    zl = jnp.dot(aggl2, wl2_ref[...], preferred_element_type=_f32) \
        + bl2_ref[...]
    lmask = jnp.where(
        lax.broadcasted_iota(jnp.int32, (1, _NATOM), 1) == _NATOM - 1,
        -1e9, 0.0)
    logits = jnp.dot(zl, wf_ref[...], preferred_element_type=_f32) \
        + bf_ref[...] + lmask
    m = jnp.max(logits, axis=1, keepdims=True)
    e = jnp.exp(logits - m)
    num = jnp.sum(e * lab, axis=1, keepdims=True)
    den = jnp.sum(e, axis=1, keepdims=True)
    lig_mask = rowid < _N
    inner = jnp.where(lig_mask, num / den, 1.0)
    lp_ref[...] += jnp.sum(jnp.log(inner), axis=0, keepdims=True)
    zlsum_ref[...] += jnp.sum(jnp.where(lig_mask, zl, 0.0), axis=0,
                              keepdims=True)
    labA_ref[...] += jnp.sum(jnp.where(lig_mask, lab, 0.0), axis=0,
                             keepdims=True)
    labB_ref[...] += jnp.sum(jnp.where(rowid < _ND, lab, 0.0), axis=0,
                             keepdims=True)

    src0 = bfs_ref[0, 0]
    dst0 = bfs_ref[1, 0]
    same = src0 == dst0
    isq = 0.70710678118654752
    x_s0 = xlf_ref[pl.ds(src0, 1), :][:, 4:]
    x_d0 = xlf_ref[pl.ds(dst0, 1), :][:, 4:]
    agg_d = jnp.where(same, x_d0, isq * x_s0 + 0.5 * x_d0)
    h_spec = jnp.maximum(
        jnp.dot(agg_d, wd1_ref[...], preferred_element_type=_f32)
        + bd1_ref[...], 0.0)
    h_src0 = jnp.maximum(
        jnp.dot(x_s0, wd1_ref[...], preferred_element_type=_f32)
        + bd1_ref[...], 0.0)
    agg2_d = jnp.where(same, h_spec, isq * h_src0 + 0.5 * h_spec)
    z_spec = jnp.dot(agg2_d, wd2_ref[...], preferred_element_type=_f32) \
        + bd2_ref[...]
    h = jnp.maximum(
        jnp.dot(lab, wd1_ref[...], preferred_element_type=_f32)
        + bd1_ref[...], 0.0)
    h = jnp.where(rowid == dst0, h_spec, h)
    zv = jnp.dot(h, wd2_ref[...], preferred_element_type=_f32) \
        + bd2_ref[...]
    zv = jnp.where(rowid == dst0, z_spec, zv)
    zv_ref[...] = zv
    zvsum_ref[...] += jnp.sum(jnp.where(rowid < _ND, zv, 0.0), axis=0,
                              keepdims=True)

    @pl.when(i == _GRID - 1)
    def _():
        hi = jnp.concatenate([zlsum_ref[...] / _N, labA_ref[...] / _N],
                             axis=1)
        hi_ref[...] = hi[0]
        ht = jnp.concatenate([zvsum_ref[...] / _ND, labB_ref[...] / _ND],
                             axis=1)
        ht_ref[...] = ht[0]


def _fin_call(bfs, aggl2, y23, invl, x_l,
              Wl2, bl2, Wf, bf, Wd1, bd1, Wd2, bd2):
    blk = lambda r, c: pl.BlockSpec((r, c), lambda i: (i, 0))
    blk3 = lambda c: pl.BlockSpec((_NC, _BM, c), lambda i: (0, i, 0))
    cst = lambda r, c: pl.BlockSpec((r, c), lambda i: (0, 0))
    return pl.pallas_call(
        _fin_body,
        grid=(_GRID,),
        in_specs=[pl.BlockSpec(memory_space=pltpu.SMEM),
                  blk3(_HD), blk3(_HD), blk(_BM, 1),
                  blk(_BM, 15),
                  cst(_N, 15),
                  cst(128, 128), cst(1, 128), cst(128, _NATOM),
                  cst(1, _NATOM),
                  cst(_NATOM, 128), cst(1, 128), cst(128, 128),
                  cst(1, 128)],
        out_specs=[blk(_BM, 128), cst(1, 1),
                   pl.BlockSpec((139,), lambda i: (0,)),
                   pl.BlockSpec((139,), lambda i: (0,))],
        out_shape=[jax.ShapeDtypeStruct((_ND, 128), _f32),
                   jax.ShapeDtypeStruct((1, 1), _f32),
                   jax.ShapeDtypeStruct((139,), _f32),
                   jax.ShapeDtypeStruct((139,), _f32)],
        scratch_shapes=[pltpu.VMEM((1, 128), _f32),
                        pltpu.VMEM((1, 128), _f32),
                        pltpu.VMEM((1, _NATOM), _f32),
                        pltpu.VMEM((1, _NATOM), _f32)],
    )(bfs, aggl2, y23, invl, x_l, x_l, Wl2, bl2[None, :], Wf,
      bf[None, :], Wd1, bd1[None, :], Wd2, bd2[None, :])


def kernel(x_p, edge_index_p, x_l, edge_index_l, bfs_init, Wp1, bp1, Wp2,
           bp2, Wl1, bl1, Wl2, bl2, Wd1, bd1, Wd2, bd2, Wf, bf):
    ep4 = edge_index_p.reshape(2, _NS, _NCHP, _CH)
    el4 = edge_index_l.reshape(2, _NS, _NCHP, _CH)

    dp, dl = _deg_call(ep4, el4)
    degp = (dp[0] + dp[1] + 1.0)[:, None]
    degl = (dl[0] + dl[1] + 1.0)[:, None]

    yp3, yl, invp, invl = _prep_call(degp, degl, x_p, x_l)

    aggp, gmat, aggl = _mid_call(yp3, ep4, invp.reshape(_NPAD), yl, el4)
    gcol = (gmat[0] + gmat[1])[:, None]

    y23, z_pocket = _mid_tc_call(aggp, yp3, invp, gcol, aggl, yl, invl,
                                 Wp1, bp1, Wp2, bp2, Wl1, bl1)

    (aggl2,) = _l2_call(y23, el4)

    z_v, lp, H_init, H_t = _fin_call(
        bfs_init, aggl2, y23, invl, x_l,
        Wl2, bl2, Wf, bf, Wd1, bd1, Wd2, bd2)

    return (lp[0, 0], z_pocket, z_v, H_init, H_t)


# speedup vs baseline: 41.7647x; 1.1088x over previous
import functools

import jax
import jax.numpy as jnp
from jax import lax
from jax.experimental import pallas as pl
from jax.experimental.pallas import tpu as pltpu
from jax.experimental.pallas import tpu_sc as plsc

_N = 10000
_E = 320000
_NPAD = 10240
_NATOM = 11
_ND = 10001

_NC, _NS, _LANES = 2, 16, 16
_CH = 100
_NCHL = _E // (_NC * _NS * _CH)
_NCHP = _E // (_NS * _CH)
_RPT = _NPAD // _NS
_HD = 64

_BM = 512
_GRID = _NPAD // _BM

_f32 = jnp.float32

_sc_mesh = plsc.VectorSubcoreMesh(
    core_axis_name="c", subcore_axis_name="s",
    num_cores=_NC, num_subcores=_NS)
_sc_params = pltpu.CompilerParams(use_tc_tiling_on_sc=False)


def _fill_1d(buf, n, value):
    v = jnp.full((_LANES,), value, _f32)

    def body(i, _):
        buf[pl.ds(i * _LANES, _LANES)] = v
        return 0

    lax.fori_loop(0, n // _LANES, body, 0)


def _fill_2d(buf, rows, cols, value):
    v = jnp.full((_LANES,), value, _f32)
    nseg = cols // _LANES

    def body(i, _):
        for k in range(nseg):
            buf[i, pl.ds(k * _LANES, _LANES)] = v
        return 0

    lax.fori_loop(0, rows, body, 0)


class _Stream:
    def __init__(self, y_hbm, gidx, sidx2, acc_s, b0, b1, gs0, gs1):
        self.y_hbm = y_hbm
        self.gidx = gidx
        self.sidx = sidx2
        self.acc = acc_s
        self.b = (b0, b1)
        self.gs = (gs0, gs1)

    def start(self, k, j):
        pltpu.async_copy(self.y_hbm.at[self.gidx.at[j]], self.b[k],
                         self.gs[k])

    def gwait(self, k):
        pltpu.make_async_copy(self.y_hbm.at[pl.ds(0, _CH)], self.b[k],
                              self.gs[k]).wait()

    def scat(self, k, j):
        pltpu.sync_copy(self.b[k], self.acc.at[self.sidx.at[j]],
                        add=True)


def _run_stream(st, nch, base):
    st.start(0, base)
    st.start(1, base + 1)

    def body(t, _):
        j = base + 2 * t
        st.gwait(0)
        st.scat(0, j)

        @pl.when(j + 2 < base + nch)
        def _():
            st.start(0, j + 2)

        st.gwait(1)
        st.scat(1, j + 1)

        @pl.when(j + 3 < base + nch)
        def _():
            st.start(1, j + 3)

        return 0

    lax.fori_loop(0, nch // 2, body, 0)


class _Stream4:
    def __init__(self, y_hbm, gidx, sidx2, acc_s, bufs, sems):
        self.y_hbm = y_hbm
        self.gidx = gidx
        self.sidx = sidx2
        self.acc = acc_s
        self.b = bufs
        self.gs = sems

    def start(self, k, j):
        pltpu.async_copy(self.y_hbm.at[self.gidx.at[j]], self.b[k],
                         self.gs[k])

    def gwait(self, k):
        pltpu.make_async_copy(self.y_hbm.at[pl.ds(0, _CH)], self.b[k],
                              self.gs[k]).wait()

    def scat(self, k, j):
        pltpu.sync_copy(self.b[k], self.acc.at[self.sidx.at[j]],
                        add=True)


def _run_stream4(st, nch):
    for k in range(4):
        st.start(k, k)

    def body(t, _):
        j = 4 * t
        for k in range(4):
            st.gwait(k)
            st.scat(k, j + k)

            @pl.when(j + k + 4 < nch)
            def _():
                st.start(k, j + k + 4)

        return 0

    lax.fori_loop(0, nch // 4, body, 0)


def _deg_body(ep_hbm, el_hbm, outp_hbm, outl_hbm,
              idx_v, ones_v, zed_v, histp_s, histl_s, sem):
    c = lax.axis_index("c")
    s = lax.axis_index("s")
    _fill_1d(zed_v, _RPT, 0.0)
    _fill_1d(ones_v, 112, 1.0)
    pltpu.sync_copy(zed_v, histp_s.at[pl.ds(s * _RPT, _RPT)])
    pltpu.sync_copy(zed_v, histl_s.at[pl.ds(s * _RPT, _RPT)])
    plsc.subcore_barrier()

    ones_sl = ones_v.at[pl.ds(0, _CH)]

    def scatter_ones(hist_s):
        def wait_one():
            pltpu.make_async_copy(ones_sl, hist_s.at[idx_v.at[0]],
                                  sem).wait()

        def body(j, _):
            @pl.when(j >= 5)
            def _():
                wait_one()

            pltpu.async_copy(ones_sl, hist_s.at[idx_v.at[j]], sem,
                             add=True)
            return 0

        lax.fori_loop(0, _NCHL, body, 0)
        for _k in range(5):
            wait_one()

    pltpu.sync_copy(ep_hbm.at[1, s, pl.ds(c * _NCHL, _NCHL)], idx_v)
    scatter_ones(histp_s)
    pltpu.sync_copy(el_hbm.at[1, s, pl.ds(c * _NCHL, _NCHL)], idx_v)
    scatter_ones(histl_s)
    plsc.subcore_barrier()
    sl = pl.ds(s * _RPT, _RPT)
    pltpu.sync_copy(histp_s.at[sl], outp_hbm.at[c, sl])
    pltpu.sync_copy(histl_s.at[sl], outl_hbm.at[c, sl])


_deg_call = pl.kernel(
    _deg_body,
    out_type=[jax.ShapeDtypeStruct((_NC, _NPAD), _f32),
              jax.ShapeDtypeStruct((_NC, _NPAD), _f32)],
    mesh=_sc_mesh,
    compiler_params=_sc_params,
    scratch_types=[
        pltpu.VMEM((_NCHL, _CH), jnp.int32),
        pltpu.VMEM((112,), _f32),
        pltpu.VMEM((_RPT,), _f32),
        pltpu.VMEM_SHARED((_NPAD,), _f32),
        pltpu.VMEM_SHARED((_NPAD,), _f32),
        pltpu.SemaphoreType.DMA,
    ])


def _zero_acc2d(gbuf, acc_s, s, cols):
    _fill_2d(gbuf, _CH, cols, 0.0)
    zsl = gbuf.at[pl.ds(0, 80)]
    for k in range(_RPT // 80):
        pltpu.sync_copy(zsl, acc_s.at[pl.ds(s * _RPT + k * 80, 80)])


def _mid_body(yp_hbm, ep_hbm, inv_hbm, yl_hbm, el_hbm,
              aggp_hbm, g_hbm, aggl_hbm,
              sidx, didx, lsidx, ldidx, rb0, rb1, vb0, vb1, lb0, lb1,
              accp_s, gacc_s, accl_s,
              rgs0, rgs1, ggs0, ggs1, lgs0, lgs1):
    c = lax.axis_index("c")
    s = lax.axis_index("s")
    _zero_acc2d(rb0, accp_s, s, _HD)
    _zero_acc2d(lb0, accl_s, s, 16)
    _fill_1d(vb0, _CH, 0.0)
    zvs = vb0.at[pl.ds(0, 80)]
    for k in range(_RPT // 80):
        pltpu.sync_copy(zvs, gacc_s.at[pl.ds(s * _RPT + k * 80, 80)])
    plsc.subcore_barrier()

    pltpu.sync_copy(ep_hbm.at[0, s], sidx)
    pltpu.sync_copy(ep_hbm.at[1, s], didx)
    pltpu.sync_copy(el_hbm.at[0, s, pl.ds(c * _NCHL, _NCHL)], lsidx)
    pltpu.sync_copy(el_hbm.at[1, s, pl.ds(c * _NCHL, _NCHL)], ldidx)

    R = _Stream(yp_hbm.at[c], sidx, didx, accp_s, rb0, rb1, rgs0, rgs1)
    G = _Stream(inv_hbm, didx, sidx, gacc_s, vb0, vb1, ggs0, ggs1)
    L = _Stream(yl_hbm, lsidx, ldidx, accl_s, lb0, lb1, lgs0, lgs1)

    gb = c * _NCHL
    R.start(0, 0)
    R.start(1, 1)
    G.start(0, gb)
    G.start(1, gb + 1)
    L.start(0, 0)
    L.start(1, 1)
    nhalf = _NCHL // 2

    def body(t, _):
        j = 2 * t
        R.gwait(0)
        R.scat(0, j)

        @pl.when(j + 2 < _NCHP)
        def _():
            R.start(0, j + 2)

        R.gwait(1)
        R.scat(1, j + 1)

        @pl.when(j + 3 < _NCHP)
        def _():
            R.start(1, j + 3)

        @pl.when(t < nhalf)
        def _():
            G.gwait(0)
            G.scat(0, gb + j)

            @pl.when(j + 2 < _NCHL)
            def _():
                G.start(0, gb + j + 2)

            G.gwait(1)
            G.scat(1, gb + j + 1)

            @pl.when(j + 3 < _NCHL)
            def _():
                G.start(1, gb + j + 3)

            L.gwait(0)
            L.scat(0, j)

            @pl.when(j + 2 < _NCHL)
            def _():
                L.start(0, j + 2)

            L.gwait(1)
            L.scat(1, j + 1)

            @pl.when(j + 3 < _NCHL)
            def _():
                L.start(1, j + 3)

        return 0

    lax.fori_loop(0, _NCHP // 2, body, 0)

    plsc.subcore_barrier()
    sl = pl.ds(s * _RPT, _RPT)
    pltpu.sync_copy(accp_s.at[sl], aggp_hbm.at[c, sl])
    pltpu.sync_copy(gacc_s.at[sl], g_hbm.at[c, sl])
    pltpu.sync_copy(accl_s.at[sl], aggl_hbm.at[c, sl])


_mid_call = pl.kernel(
    _mid_body,
    out_type=[jax.ShapeDtypeStruct((_NC, _NPAD, _HD), _f32),
              jax.ShapeDtypeStruct((_NC, _NPAD), _f32),
              jax.ShapeDtypeStruct((_NC, _NPAD, 16), _f32)],
    mesh=_sc_mesh,
    compiler_params=_sc_params,
    scratch_types=[
        pltpu.VMEM((_NCHP, _CH), jnp.int32),
        pltpu.VMEM((_NCHP, _CH), jnp.int32),
        pltpu.VMEM((_NCHL, _CH), jnp.int32),
        pltpu.VMEM((_NCHL, _CH), jnp.int32),
        pltpu.VMEM((_CH, _HD), _f32),
        pltpu.VMEM((_CH, _HD), _f32),
        pltpu.VMEM((_CH,), _f32),
        pltpu.VMEM((_CH,), _f32),
        pltpu.VMEM((_CH, 16), _f32),
        pltpu.VMEM((_CH, 16), _f32),
        pltpu.VMEM_SHARED((_NPAD, _HD), _f32),
        pltpu.VMEM_SHARED((_NPAD,), _f32),
        pltpu.VMEM_SHARED((_NPAD, 16), _f32),
    ] + [pltpu.SemaphoreType.DMA] * 6)


def _l2_body(y2_hbm, el_hbm, agg_hbm,
             sidx, didx, gb0, gb1, gb2, gb3, acc_s, gs0, gs1, gs2, gs3):
    c = lax.axis_index("c")
    s = lax.axis_index("s")
    _zero_acc2d(gb0, acc_s, s, _HD)
    plsc.subcore_barrier()
    pltpu.sync_copy(el_hbm.at[0, s], sidx)
    pltpu.sync_copy(el_hbm.at[1, s], didx)
    st = _Stream4(y2_hbm.at[c], sidx, didx, acc_s,
                  (gb0, gb1, gb2, gb3), (gs0, gs1, gs2, gs3))
    _run_stream4(st, _NCHP)
    plsc.subcore_barrier()
    sl = pl.ds(s * _RPT, _RPT)
    pltpu.sync_copy(acc_s.at[sl], agg_hbm.at[c, sl])


_l2_call = pl.kernel(
    _l2_body,
    out_type=[jax.ShapeDtypeStruct((_NC, _NPAD, _HD), _f32)],
    mesh=_sc_mesh,
    compiler_params=_sc_params,
    scratch_types=[
        pltpu.VMEM((_NCHP, _CH), jnp.int32),
        pltpu.VMEM((_NCHP, _CH), jnp.int32),
        pltpu.VMEM((_CH, _HD), _f32),
        pltpu.VMEM((_CH, _HD), _f32),
        pltpu.VMEM((_CH, _HD), _f32),
        pltpu.VMEM((_CH, _HD), _f32),
        pltpu.VMEM_SHARED((_NPAD, _HD), _f32),
        pltpu.SemaphoreType.DMA,
        pltpu.SemaphoreType.DMA,
        pltpu.SemaphoreType.DMA,
        pltpu.SemaphoreType.DMA,
    ])


def _prep_body(degp_ref, degl_ref, xp_ref, xl_ref,
               yp_ref, yl_ref, invp_ref, invl_ref):
    invp = lax.rsqrt(degp_ref[...])
    invl = lax.rsqrt(degl_ref[...])
    invp_ref[...] = invp
    invl_ref[...] = invl
    yp = xp_ref[...] * invp
    yp_ref[0, :, :] = yp[:, :_HD]
    yp_ref[1, :, :] = yp[:, _HD:]
    xl16 = jnp.concatenate(
        [xl_ref[...], jnp.zeros((_BM, 1), _f32)], axis=1)
    yl_ref[...] = xl16 * invl


def _prep_call(degp, degl, x_p, x_l):
    blkc = lambda c: pl.BlockSpec((_NC, _BM, c), lambda i: (0, i, 0))
    return pl.pallas_call(
        _prep_body,
        grid=(_GRID,),
        in_specs=[pl.BlockSpec((_BM, 1), lambda i: (i, 0)),
                  pl.BlockSpec((_BM, 1), lambda i: (i, 0)),
                  pl.BlockSpec((_BM, 128), lambda i: (i, 0)),
                  pl.BlockSpec((_BM, 15), lambda i: (i, 0))],
        out_specs=[blkc(_HD),
                   pl.BlockSpec((_BM, 16), lambda i: (i, 0)),
                   pl.BlockSpec((_BM, 1), lambda i: (i, 0)),
                   pl.BlockSpec((_BM, 1), lambda i: (i, 0))],
        out_shape=[jax.ShapeDtypeStruct((_NC, _NPAD, _HD), _f32),
                   jax.ShapeDtypeStruct((_NPAD, 16), _f32),
                   jax.ShapeDtypeStruct((_NPAD, 1), _f32),
                   jax.ShapeDtypeStruct((_NPAD, 1), _f32)],
    )(degp, degl, x_p, x_l)


def _mid_tc_body(ap_ref, yp_ref, invp_ref, g_ref,
                 al_ref, yl_ref, invl_ref,
                 wp1_ref, bp1_ref, wp2_ref, bp2_ref, wl1_ref, bl1_ref,
                 y2_ref, zp_ref, sacc_ref):
    i = pl.program_id(0)
    invp = invp_ref[...]
    aggp = invp * jnp.concatenate(
        [ap_ref[0] + yp_ref[0], ap_ref[1] + yp_ref[1]], axis=1)
    h = jnp.maximum(
        jnp.dot(aggp, wp1_ref[...], preferred_element_type=_f32)
        + bp1_ref[...], 0.0)
    rowid = lax.broadcasted_iota(jnp.int32, (_BM, 1), 0) + i * _BM
    cvec = jnp.where(rowid < _N, invp * (g_ref[...] + invp), 0.0)

    @pl.when(i == 0)
    def _():
        sacc_ref[...] = jnp.zeros_like(sacc_ref)

    sacc_ref[...] += jnp.sum(cvec * h, axis=0, keepdims=True)

    invl = invl_ref[...]
    aggl = invl * (al_ref[0] + al_ref[1] + yl_ref[...])
    wl1p = jnp.concatenate([wl1_ref[...], jnp.zeros((1, 128), _f32)],
                           axis=0)
    hl = jnp.maximum(
        jnp.dot(aggl, wl1p, preferred_element_type=_f32)
        + bl1_ref[...], 0.0)
    y2 = invl * hl
    y2_ref[0, :, :] = y2[:, :_HD]
    y2_ref[1, :, :] = y2[:, _HD:]

    @pl.when(i == _GRID - 1)
    def _():
        zp = jnp.dot(sacc_ref[...] / _N, wp2_ref[...],
                     preferred_element_type=_f32) + bp2_ref[...]
        zp_ref[...] = zp[0]


def _mid_tc_call(aggp, yp3, invp, gmat, aggl, yl, invl,
                 Wp1, bp1, Wp2, bp2, Wl1, bl1):
    blk = lambda r, c: pl.BlockSpec((r, c), lambda i: (i, 0))
    blk3 = lambda c: pl.BlockSpec((_NC, _BM, c), lambda i: (0, i, 0))
    cst = lambda r, c: pl.BlockSpec((r, c), lambda i: (0, 0))
    return pl.pallas_call(
        _mid_tc_body,
        grid=(_GRID,),
        in_specs=[blk3(_HD), blk3(_HD), blk(_BM, 1), blk(_BM, 1),
                  blk3(16), blk(_BM, 16), blk(_BM, 1),
                  cst(128, 128), cst(1, 128), cst(128, 128), cst(1, 128),
                  cst(15, 128), cst(1, 128)],
        out_specs=[blk3(_HD),
                   pl.BlockSpec((128,), lambda i: (0,))],
        out_shape=[jax.ShapeDtypeStruct((_NC, _NPAD, _HD), _f32),
                   jax.ShapeDtypeStruct((128,), _f32)],
        scratch_shapes=[pltpu.VMEM((1, 128), _f32)],
    )(aggp, yp3, invp, gmat, aggl, yl, invl,
      Wp1, bp1[None, :], Wp2, bp2[None, :], Wl1, bl1[None, :])


def _dec_body(bfs_ref, xl_ref, xlf_ref,
              wd1_ref, bd1_ref, wd2_ref, bd2_ref,
              zv_ref, ht_ref, zvsum_ref, labB_ref):
    i = pl.program_id(0)
    rowid = lax.broadcasted_iota(jnp.int32, (_BM, 1), 0) + i * _BM

    @pl.when(i == 0)
    def _():
        zvsum_ref[...] = jnp.zeros_like(zvsum_ref)
        labB_ref[...] = jnp.zeros_like(labB_ref)

    stop_row = jnp.where(
        lax.broadcasted_iota(jnp.int32, (1, _NATOM), 1) == _NATOM - 1,
        1.0, 0.0)
    lab = jnp.where(rowid < _N, xl_ref[...][:, 4:], 0.0)
    lab = jnp.where(rowid == _N, stop_row, lab)
    labB_ref[...] += jnp.sum(jnp.where(rowid < _ND, lab, 0.0), axis=0,
                             keepdims=True)

    src0 = bfs_ref[0, 0]
    dst0 = bfs_ref[1, 0]
    same = src0 == dst0
    isq = 0.70710678118654752
    x_s0 = xlf_ref[pl.ds(src0, 1), :][:, 4:]
    x_d0 = xlf_ref[pl.ds(dst0, 1), :][:, 4:]
    agg_d = jnp.where(same, x_d0, isq * x_s0 + 0.5 * x_d0)
    h_spec = jnp.maximum(
        jnp.dot(agg_d, wd1_ref[...], preferred_element_type=_f32)
        + bd1_ref[...], 0.0)
    h_src0 = jnp.maximum(
        jnp.dot(x_s0, wd1_ref[...], preferred_element_type=_f32)
        + bd1_ref[...], 0.0)
    agg2_d = jnp.where(same, h_spec, isq * h_src0 + 0.5 * h_spec)
    z_spec = jnp.dot(agg2_d, wd2_ref[...], preferred_element_type=_f32) \
        + bd2_ref[...]
    h = jnp.maximum(
        jnp.dot(lab, wd1_ref[...], preferred_element_type=_f32)
        + bd1_ref[...], 0.0)
    h = jnp.where(rowid == dst0, h_spec, h)
    zv = jnp.dot(h, wd2_ref[...], preferred_element_type=_f32) \
        + bd2_ref[...]
    zv = jnp.where(rowid == dst0, z_spec, zv)
    zv_ref[...] = zv
    zvsum_ref[...] += jnp.sum(jnp.where(rowid < _ND, zv, 0.0), axis=0,
                              keepdims=True)

    @pl.when(i == _GRID - 1)
    def _():
        ht = jnp.concatenate([zvsum_ref[...] / _ND, labB_ref[...] / _ND],
                             axis=1)
        ht_ref[...] = ht[0]


def _dec_call(bfs, x_l, Wd1, bd1, Wd2, bd2):
    blk = lambda r, c: pl.BlockSpec((r, c), lambda i: (i, 0))
    cst = lambda r, c: pl.BlockSpec((r, c), lambda i: (0, 0))
    return pl.pallas_call(
        _dec_body,
        grid=(_GRID,),
        in_specs=[pl.BlockSpec(memory_space=pltpu.SMEM),
                  blk(_BM, 15),
                  cst(_N, 15),
                  cst(_NATOM, 128), cst(1, 128), cst(128, 128),
                  cst(1, 128)],
        out_specs=[blk(_BM, 128),
                   pl.BlockSpec((139,), lambda i: (0,))],
        out_shape=[jax.ShapeDtypeStruct((_ND, 128), _f32),
                   jax.ShapeDtypeStruct((139,), _f32)],
        scratch_shapes=[pltpu.VMEM((1, 128), _f32),
                        pltpu.VMEM((1, _NATOM), _f32)],
    )(bfs, x_l, x_l, Wd1, bd1[None, :], Wd2, bd2[None, :])


def _lig_body(q_ref, y2_ref, invl_ref, xl_ref,
              wl2_ref, bl2_ref, wf_ref, bf_ref,
              lp_ref, hi_ref, zlsum_ref, labA_ref):
    i = pl.program_id(0)
    rowid = lax.broadcasted_iota(jnp.int32, (_BM, 1), 0) + i * _BM

    @pl.when(i == 0)
    def _():
        lp_ref[...] = jnp.zeros_like(lp_ref)
        zlsum_ref[...] = jnp.zeros_like(zlsum_ref)
        labA_ref[...] = jnp.zeros_like(labA_ref)

    lab = jnp.where(rowid < _N, xl_ref[...][:, 4:], 0.0)
    invl = invl_ref[...]
    aggl2 = invl * jnp.concatenate(
        [q_ref[0] + y2_ref[0], q_ref[1] + y2_ref[1]], axis=1)
    zl = jnp.dot(aggl2, wl2_ref[...], preferred_element_type=_f32) \
        + bl2_ref[...]
    lmask = jnp.where(
        lax.broadcasted_iota(jnp.int32, (1, _NATOM), 1) == _NATOM - 1,
        -1e9, 0.0)
    logits = jnp.dot(zl, wf_ref[...], preferred_element_type=_f32) \
        + bf_ref[...] + lmask
    m = jnp.max(logits, axis=1, keepdims=True)
    e = jnp.exp(logits - m)
    num = jnp.sum(e * lab, axis=1, keepdims=True)
    den = jnp.sum(e, axis=1, keepdims=True)
    lig_mask = rowid < _N
    inner = jnp.where(lig_mask, num / den, 1.0)
    lp_ref[...] += jnp.sum(jnp.log(inner), axis=0, keepdims=True)
    zlsum_ref[...] += jnp.sum(jnp.where(lig_mask, zl, 0.0), axis=0,
                              keepdims=True)
    labA_ref[...] += jnp.sum(jnp.where(lig_mask, lab, 0.0), axis=0,
                             keepdims=True)

    @pl.when(i == _GRID - 1)
    def _():
        hi = jnp.concatenate([zlsum_ref[...] / _N, labA_ref[...] / _N],
                             axis=1)
        hi_ref[...] = hi[0]


def _lig_call(aggl2, y23, invl, x_l, Wl2, bl2, Wf, bf):
    blk = lambda r, c: pl.BlockSpec((r, c), lambda i: (i, 0))
    blk3 = lambda c: pl.BlockSpec((_NC, _BM, c), lambda i: (0, i, 0))
    cst = lambda r, c: pl.BlockSpec((r, c), lambda i: (0, 0))
    return pl.pallas_call(
        _lig_body,
        grid=(_GRID,),
        in_specs=[blk3(_HD), blk3(_HD), blk(_BM, 1), blk(_BM, 15),
                  cst(128, 128), cst(1, 128), cst(128, _NATOM),
                  cst(1, _NATOM)],
        out_specs=[cst(1, 1),
                   pl.BlockSpec((139,), lambda i: (0,))],
        out_shape=[jax.ShapeDtypeStruct((1, 1), _f32),
                   jax.ShapeDtypeStruct((139,), _f32)],
        scratch_shapes=[pltpu.VMEM((1, 128), _f32),
                        pltpu.VMEM((1, _NATOM), _f32)],
    )(aggl2, y23, invl, x_l, Wl2, bl2[None, :], Wf, bf[None, :])


def kernel(x_p, edge_index_p, x_l, edge_index_l, bfs_init, Wp1, bp1, Wp2,
           bp2, Wl1, bl1, Wl2, bl2, Wd1, bd1, Wd2, bd2, Wf, bf):
    ep4 = edge_index_p.reshape(2, _NS, _NCHP, _CH)
    el4 = edge_index_l.reshape(2, _NS, _NCHP, _CH)

    z_v, H_t = _dec_call(bfs_init, x_l, Wd1, bd1, Wd2, bd2)

    dp, dl = _deg_call(ep4, el4)
    degp = (dp[0] + dp[1] + 1.0)[:, None]
    degl = (dl[0] + dl[1] + 1.0)[:, None]

    yp3, yl, invp, invl = _prep_call(degp, degl, x_p, x_l)

    aggp, gmat, aggl = _mid_call(yp3, ep4, invp.reshape(_NPAD), yl, el4)
    gcol = (gmat[0] + gmat[1])[:, None]

    y23, z_pocket = _mid_tc_call(aggp, yp3, invp, gcol, aggl, yl, invl,
                                 Wp1, bp1, Wp2, bp2, Wl1, bl1)

    (aggl2,) = _l2_call(y23, el4)

    lp, H_init = _lig_call(aggl2, y23, invl, x_l, Wl2, bl2, Wf, bf)

    return (lp[0, 0], z_pocket, z_v, H_init, H_t)


# speedup vs baseline: 44.8595x; 1.0741x over previous
import functools

import jax
import jax.numpy as jnp
from jax import lax
from jax.experimental import pallas as pl
from jax.experimental.pallas import tpu as pltpu
from jax.experimental.pallas import tpu_sc as plsc

_N = 10000
_E = 320000
_NPAD = 10240
_NATOM = 11
_ND = 10001

_NC, _NS, _LANES = 2, 16, 16
_CH = 100
_NCHL = _E // (_NC * _NS * _CH)
_NCHP = _E // (_NS * _CH)
_RPT = _NPAD // _NS
_HD = 64

_BM = 512
_GRID = _NPAD // _BM

_f32 = jnp.float32

_sc_mesh = plsc.VectorSubcoreMesh(
    core_axis_name="c", subcore_axis_name="s",
    num_cores=_NC, num_subcores=_NS)
_sc_params = pltpu.CompilerParams(use_tc_tiling_on_sc=False)


def _fill_1d(buf, n, value):
    v = jnp.full((_LANES,), value, _f32)

    def body(i, _):
        buf[pl.ds(i * _LANES, _LANES)] = v
        return 0

    lax.fori_loop(0, n // _LANES, body, 0)


def _fill_2d(buf, rows, cols, value):
    v = jnp.full((_LANES,), value, _f32)
    nseg = cols // _LANES

    def body(i, _):
        for k in range(nseg):
            buf[i, pl.ds(k * _LANES, _LANES)] = v
        return 0

    lax.fori_loop(0, rows, body, 0)


class _Stream:
    def __init__(self, y_hbm, gidx, sidx2, acc_s, b0, b1, gs0, gs1):
        self.y_hbm = y_hbm
        self.gidx = gidx
        self.sidx = sidx2
        self.acc = acc_s
        self.b = (b0, b1)
        self.gs = (gs0, gs1)

    def start(self, k, j):
        pltpu.async_copy(self.y_hbm.at[self.gidx.at[j]], self.b[k],
                         self.gs[k])

    def gwait(self, k):
        pltpu.make_async_copy(self.y_hbm.at[pl.ds(0, _CH)], self.b[k],
                              self.gs[k]).wait()

    def scat(self, k, j):
        pltpu.sync_copy(self.b[k], self.acc.at[self.sidx.at[j]],
                        add=True)


def _run_stream(st, nch, base):
    st.start(0, base)
    st.start(1, base + 1)

    def body(t, _):
        j = base + 2 * t
        st.gwait(0)
        st.scat(0, j)

        @pl.when(j + 2 < base + nch)
        def _():
            st.start(0, j + 2)

        st.gwait(1)
        st.scat(1, j + 1)

        @pl.when(j + 3 < base + nch)
        def _():
            st.start(1, j + 3)

        return 0

    lax.fori_loop(0, nch // 2, body, 0)


class _Stream4:
    def __init__(self, y_hbm, gidx, sidx2, acc_s, bufs, sems):
        self.y_hbm = y_hbm
        self.gidx = gidx
        self.sidx = sidx2
        self.acc = acc_s
        self.b = bufs
        self.gs = sems

    def start(self, k, j):
        pltpu.async_copy(self.y_hbm.at[self.gidx.at[j]], self.b[k],
                         self.gs[k])

    def gwait(self, k):
        pltpu.make_async_copy(self.y_hbm.at[pl.ds(0, _CH)], self.b[k],
                              self.gs[k]).wait()

    def scat(self, k, j):
        pltpu.sync_copy(self.b[k], self.acc.at[self.sidx.at[j]],
                        add=True)


def _run_stream4(st, nch):
    for k in range(4):
        st.start(k, k)

    def body(t, _):
        j = 4 * t
        for k in range(4):
            st.gwait(k)
            st.scat(k, j + k)

            @pl.when(j + k + 4 < nch)
            def _():
                st.start(k, j + k + 4)

        return 0

    lax.fori_loop(0, nch // 4, body, 0)


def _deg_body(ep_hbm, el_hbm, outp_hbm, outl_hbm,
              idx_v, ones_v, zed_v, histp_s, histl_s, sem):
    c = lax.axis_index("c")
    s = lax.axis_index("s")
    _fill_1d(zed_v, _RPT, 0.0)
    _fill_1d(ones_v, 112, 1.0)
    pltpu.sync_copy(zed_v, histp_s.at[pl.ds(s * _RPT, _RPT)])
    pltpu.sync_copy(zed_v, histl_s.at[pl.ds(s * _RPT, _RPT)])
    plsc.subcore_barrier()

    ones_sl = ones_v.at[pl.ds(0, _CH)]

    def scatter_ones(hist_s):
        def wait_one():
            pltpu.make_async_copy(ones_sl, hist_s.at[idx_v.at[0]],
                                  sem).wait()

        def body(j, _):
            @pl.when(j >= 5)
            def _():
                wait_one()

            pltpu.async_copy(ones_sl, hist_s.at[idx_v.at[j]], sem,
                             add=True)
            return 0

        lax.fori_loop(0, _NCHL, body, 0)
        for _k in range(5):
            wait_one()

    pltpu.sync_copy(ep_hbm.at[1, s, pl.ds(c * _NCHL, _NCHL)], idx_v)
    scatter_ones(histp_s)
    pltpu.sync_copy(el_hbm.at[1, s, pl.ds(c * _NCHL, _NCHL)], idx_v)
    scatter_ones(histl_s)
    plsc.subcore_barrier()
    sl = pl.ds(s * _RPT, _RPT)
    pltpu.sync_copy(histp_s.at[sl], outp_hbm.at[c, sl])
    pltpu.sync_copy(histl_s.at[sl], outl_hbm.at[c, sl])


_deg_call = pl.kernel(
    _deg_body,
    out_type=[jax.ShapeDtypeStruct((_NC, _NPAD), _f32),
              jax.ShapeDtypeStruct((_NC, _NPAD), _f32)],
    mesh=_sc_mesh,
    compiler_params=_sc_params,
    scratch_types=[
        pltpu.VMEM((_NCHL, _CH), jnp.int32),
        pltpu.VMEM((112,), _f32),
        pltpu.VMEM((_RPT,), _f32),
        pltpu.VMEM_SHARED((_NPAD,), _f32),
        pltpu.VMEM_SHARED((_NPAD,), _f32),
        pltpu.SemaphoreType.DMA,
    ])


def _zero_acc2d(gbuf, acc_s, s, cols):
    _fill_2d(gbuf, _CH, cols, 0.0)
    zsl = gbuf.at[pl.ds(0, 80)]
    for k in range(_RPT // 80):
        pltpu.sync_copy(zsl, acc_s.at[pl.ds(s * _RPT + k * 80, 80)])


def _mid_body(yp_hbm, ep_hbm, inv_hbm, yl_hbm, el_hbm,
              aggp_hbm, g_hbm, aggl_hbm,
              sidx, didx, lsidx, ldidx,
              rb0, rb1, rb2, rb3, vb0, vb1, vb2, vb3,
              lb0, lb1, lb2, lb3,
              accp_s, gacc_s, accl_s,
              rs0, rs1, rs2, rs3, gs0, gs1, gs2, gs3,
              ls0, ls1, ls2, ls3):
    c = lax.axis_index("c")
    s = lax.axis_index("s")
    _zero_acc2d(rb0, accp_s, s, _HD)
    _zero_acc2d(lb0, accl_s, s, 16)
    _fill_1d(vb0, 96, 0.0)
    zvs = vb0.at[pl.ds(0, 80)]
    for k in range(_RPT // 80):
        pltpu.sync_copy(zvs, gacc_s.at[pl.ds(s * _RPT + k * 80, 80)])
    plsc.subcore_barrier()

    pltpu.sync_copy(el_hbm.at[0, s, pl.ds(c * _NCHL, _NCHL)], lsidx)
    pltpu.sync_copy(el_hbm.at[1, s, pl.ds(c * _NCHL, _NCHL)], ldidx)

    R = _Stream4(yp_hbm.at[c], sidx, didx, accp_s,
                 (rb0, rb1, rb2, rb3), (rs0, rs1, rs2, rs3))
    G = _Stream4(inv_hbm, didx, sidx, gacc_s,
                 (vb0, vb1, vb2, vb3), (gs0, gs1, gs2, gs3))
    L = _Stream4(yl_hbm, lsidx, ldidx, accl_s,
                 (lb0, lb1, lb2, lb3), (ls0, ls1, ls2, ls3))

    for p in range(2):
        pltpu.sync_copy(ep_hbm.at[0, s, pl.ds(p * _NCHL, _NCHL)], sidx)
        pltpu.sync_copy(ep_hbm.at[1, s, pl.ds(p * _NCHL, _NCHL)], didx)
        for k in range(4):
            R.start(k, k)

        @pl.when(c == p)
        def _():
            for k in range(4):
                G.start(k, k)

        if p == 0:
            for k in range(4):
                L.start(k, k)

        def body(t, _):
            j = 4 * t
            for k in range(4):
                R.gwait(k)
                R.scat(k, j + k)

                @pl.when(j + k + 4 < _NCHL)
                def _():
                    R.start(k, j + k + 4)

                @pl.when(c == p)
                def _():
                    G.gwait(k)
                    G.scat(k, j + k)

                    @pl.when(j + k + 4 < _NCHL)
                    def _():
                        G.start(k, j + k + 4)

                if p == 0:
                    L.gwait(k)
                    L.scat(k, j + k)

                    @pl.when(j + k + 4 < _NCHL)
                    def _():
                        L.start(k, j + k + 4)

            return 0

        lax.fori_loop(0, _NCHL // 4, body, 0)

    plsc.subcore_barrier()
    sl = pl.ds(s * _RPT, _RPT)
    pltpu.sync_copy(accp_s.at[sl], aggp_hbm.at[c, sl])
    pltpu.sync_copy(gacc_s.at[sl], g_hbm.at[c, sl])
    pltpu.sync_copy(accl_s.at[sl], aggl_hbm.at[c, sl])


_mid_call = pl.kernel(
    _mid_body,
    out_type=[jax.ShapeDtypeStruct((_NC, _NPAD, _HD), _f32),
              jax.ShapeDtypeStruct((_NC, _NPAD), _f32),
              jax.ShapeDtypeStruct((_NC, _NPAD, 16), _f32)],
    mesh=_sc_mesh,
    compiler_params=_sc_params,
    scratch_types=[
        pltpu.VMEM((_NCHL, _CH), jnp.int32),
        pltpu.VMEM((_NCHL, _CH), jnp.int32),
        pltpu.VMEM((_NCHL, _CH), jnp.int32),
        pltpu.VMEM((_NCHL, _CH), jnp.int32),
        pltpu.VMEM((_CH, _HD), _f32),
        pltpu.VMEM((_CH, _HD), _f32),
        pltpu.VMEM((_CH, _HD), _f32),
        pltpu.VMEM((_CH, _HD), _f32),
        pltpu.VMEM((_CH,), _f32),
        pltpu.VMEM((_CH,), _f32),
        pltpu.VMEM((_CH,), _f32),
        pltpu.VMEM((_CH,), _f32),
        pltpu.VMEM((_CH, 16), _f32),
        pltpu.VMEM((_CH, 16), _f32),
        pltpu.VMEM((_CH, 16), _f32),
        pltpu.VMEM((_CH, 16), _f32),
        pltpu.VMEM_SHARED((_NPAD, _HD), _f32),
        pltpu.VMEM_SHARED((_NPAD,), _f32),
        pltpu.VMEM_SHARED((_NPAD, 16), _f32),
    ] + [pltpu.SemaphoreType.DMA] * 12)


def _l2_body(y2_hbm, el_hbm, agg_hbm,
             sidx, didx, gb0, gb1, gb2, gb3, acc_s, gs0, gs1, gs2, gs3):
    c = lax.axis_index("c")
    s = lax.axis_index("s")
    _zero_acc2d(gb0, acc_s, s, _HD)
    plsc.subcore_barrier()
    pltpu.sync_copy(el_hbm.at[0, s], sidx)
    pltpu.sync_copy(el_hbm.at[1, s], didx)
    st = _Stream4(y2_hbm.at[c], sidx, didx, acc_s,
                  (gb0, gb1, gb2, gb3), (gs0, gs1, gs2, gs3))
    _run_stream4(st, _NCHP)
    plsc.subcore_barrier()
    sl = pl.ds(s * _RPT, _RPT)
    pltpu.sync_copy(acc_s.at[sl], agg_hbm.at[c, sl])


_l2_call = pl.kernel(
    _l2_body,
    out_type=[jax.ShapeDtypeStruct((_NC, _NPAD, _HD), _f32)],
    mesh=_sc_mesh,
    compiler_params=_sc_params,
    scratch_types=[
        pltpu.VMEM((_NCHP, _CH), jnp.int32),
        pltpu.VMEM((_NCHP, _CH), jnp.int32),
        pltpu.VMEM((_CH, _HD), _f32),
        pltpu.VMEM((_CH, _HD), _f32),
        pltpu.VMEM((_CH, _HD), _f32),
        pltpu.VMEM((_CH, _HD), _f32),
        pltpu.VMEM_SHARED((_NPAD, _HD), _f32),
        pltpu.SemaphoreType.DMA,
        pltpu.SemaphoreType.DMA,
        pltpu.SemaphoreType.DMA,
        pltpu.SemaphoreType.DMA,
    ])


def _prep_body(degp_ref, degl_ref, xp_ref, xl_ref,
               yp_ref, yl_ref, invp_ref, invl_ref):
    invp = lax.rsqrt(degp_ref[...])
    invl = lax.rsqrt(degl_ref[...])
    invp_ref[...] = invp
    invl_ref[...] = invl
    yp = xp_ref[...] * invp
    yp_ref[0, :, :] = yp[:, :_HD]
    yp_ref[1, :, :] = yp[:, _HD:]
    xl16 = jnp.concatenate(
        [xl_ref[...], jnp.zeros((_BM, 1), _f32)], axis=1)
    yl_ref[...] = xl16 * invl


def _prep_call(degp, degl, x_p, x_l):
    blkc = lambda c: pl.BlockSpec((_NC, _BM, c), lambda i: (0, i, 0))
    return pl.pallas_call(
        _prep_body,
        grid=(_GRID,),
        in_specs=[pl.BlockSpec((_BM, 1), lambda i: (i, 0)),
                  pl.BlockSpec((_BM, 1), lambda i: (i, 0)),
                  pl.BlockSpec((_BM, 128), lambda i: (i, 0)),
                  pl.BlockSpec((_BM, 15), lambda i: (i, 0))],
        out_specs=[blkc(_HD),
                   pl.BlockSpec((_BM, 16), lambda i: (i, 0)),
                   pl.BlockSpec((_BM, 1), lambda i: (i, 0)),
                   pl.BlockSpec((_BM, 1), lambda i: (i, 0))],
        out_shape=[jax.ShapeDtypeStruct((_NC, _NPAD, _HD), _f32),
                   jax.ShapeDtypeStruct((_NPAD, 16), _f32),
                   jax.ShapeDtypeStruct((_NPAD, 1), _f32),
                   jax.ShapeDtypeStruct((_NPAD, 1), _f32)],
    )(degp, degl, x_p, x_l)


def _mid_tc_body(ap_ref, yp_ref, invp_ref, g_ref,
                 al_ref, yl_ref, invl_ref,
                 wp1_ref, bp1_ref, wp2_ref, bp2_ref, wl1_ref, bl1_ref,
                 y2_ref, zp_ref, sacc_ref):
    i = pl.program_id(0)
    invp = invp_ref[...]
    aggp = invp * jnp.concatenate(
        [ap_ref[0] + yp_ref[0], ap_ref[1] + yp_ref[1]], axis=1)
    h = jnp.maximum(
        jnp.dot(aggp, wp1_ref[...], preferred_element_type=_f32)
        + bp1_ref[...], 0.0)
    rowid = lax.broadcasted_iota(jnp.int32, (_BM, 1), 0) + i * _BM
    cvec = jnp.where(rowid < _N, invp * (g_ref[...] + invp), 0.0)

    @pl.when(i == 0)
    def _():
        sacc_ref[...] = jnp.zeros_like(sacc_ref)

    sacc_ref[...] += jnp.sum(cvec * h, axis=0, keepdims=True)

    invl = invl_ref[...]
    aggl = invl * (al_ref[0] + al_ref[1] + yl_ref[...])
    wl1p = jnp.concatenate([wl1_ref[...], jnp.zeros((1, 128), _f32)],
                           axis=0)
    hl = jnp.maximum(
        jnp.dot(aggl, wl1p, preferred_element_type=_f32)
        + bl1_ref[...], 0.0)
    y2 = invl * hl
    y2_ref[0, :, :] = y2[:, :_HD]
    y2_ref[1, :, :] = y2[:, _HD:]

    @pl.when(i == _GRID - 1)
    def _():
        zp = jnp.dot(sacc_ref[...] / _N, wp2_ref[...],
                     preferred_element_type=_f32) + bp2_ref[...]
        zp_ref[...] = zp[0]


def _mid_tc_call(aggp, yp3, invp, gmat, aggl, yl, invl,
                 Wp1, bp1, Wp2, bp2, Wl1, bl1):
    blk = lambda r, c: pl.BlockSpec((r, c), lambda i: (i, 0))
    blk3 = lambda c: pl.BlockSpec((_NC, _BM, c), lambda i: (0, i, 0))
    cst = lambda r, c: pl.BlockSpec((r, c), lambda i: (0, 0))
    return pl.pallas_call(
        _mid_tc_body,
        grid=(_GRID,),
        in_specs=[blk3(_HD), blk3(_HD), blk(_BM, 1), blk(_BM, 1),
                  blk3(16), blk(_BM, 16), blk(_BM, 1),
                  cst(128, 128), cst(1, 128), cst(128, 128), cst(1, 128),
                  cst(15, 128), cst(1, 128)],
        out_specs=[blk3(_HD),
                   pl.BlockSpec((128,), lambda i: (0,))],
        out_shape=[jax.ShapeDtypeStruct((_NC, _NPAD, _HD), _f32),
                   jax.ShapeDtypeStruct((128,), _f32)],
        scratch_shapes=[pltpu.VMEM((1, 128), _f32)],
    )(aggp, yp3, invp, gmat, aggl, yl, invl,
      Wp1, bp1[None, :], Wp2, bp2[None, :], Wl1, bl1[None, :])


def _dec_body(bfs_ref, xl_ref, xlf_ref,
              wd1_ref, bd1_ref, wd2_ref, bd2_ref,
              zv_ref, ht_ref, zvsum_ref, labB_ref):
    i = pl.program_id(0)
    rowid = lax.broadcasted_iota(jnp.int32, (_BM, 1), 0) + i * _BM

    @pl.when(i == 0)
    def _():
        zvsum_ref[...] = jnp.zeros_like(zvsum_ref)
        labB_ref[...] = jnp.zeros_like(labB_ref)

    stop_row = jnp.where(
        lax.broadcasted_iota(jnp.int32, (1, _NATOM), 1) == _NATOM - 1,
        1.0, 0.0)
    lab = jnp.where(rowid < _N, xl_ref[...][:, 4:], 0.0)
    lab = jnp.where(rowid == _N, stop_row, lab)
    labB_ref[...] += jnp.sum(jnp.where(rowid < _ND, lab, 0.0), axis=0,
                             keepdims=True)

    src0 = bfs_ref[0, 0]
    dst0 = bfs_ref[1, 0]
    same = src0 == dst0
    isq = 0.70710678118654752
    x_s0 = xlf_ref[pl.ds(src0, 1), :][:, 4:]
    x_d0 = xlf_ref[pl.ds(dst0, 1), :][:, 4:]
    agg_d = jnp.where(same, x_d0, isq * x_s0 + 0.5 * x_d0)
    h_spec = jnp.maximum(
        jnp.dot(agg_d, wd1_ref[...], preferred_element_type=_f32)
        + bd1_ref[...], 0.0)
    h_src0 = jnp.maximum(
        jnp.dot(x_s0, wd1_ref[...], preferred_element_type=_f32)
        + bd1_ref[...], 0.0)
    agg2_d = jnp.where(same, h_spec, isq * h_src0 + 0.5 * h_spec)
    z_spec = jnp.dot(agg2_d, wd2_ref[...], preferred_element_type=_f32) \
        + bd2_ref[...]
    h = jnp.maximum(
        jnp.dot(lab, wd1_ref[...], preferred_element_type=_f32)
        + bd1_ref[...], 0.0)
    h = jnp.where(rowid == dst0, h_spec, h)
    zv = jnp.dot(h, wd2_ref[...], preferred_element_type=_f32) \
        + bd2_ref[...]
    zv = jnp.where(rowid == dst0, z_spec, zv)
    zv_ref[...] = zv
    zvsum_ref[...] += jnp.sum(jnp.where(rowid < _ND, zv, 0.0), axis=0,
                              keepdims=True)

    @pl.when(i == _GRID - 1)
    def _():
        ht = jnp.concatenate([zvsum_ref[...] / _ND, labB_ref[...] / _ND],
                             axis=1)
        ht_ref[...] = ht[0]


def _dec_call(bfs, x_l, Wd1, bd1, Wd2, bd2):
    blk = lambda r, c: pl.BlockSpec((r, c), lambda i: (i, 0))
    cst = lambda r, c: pl.BlockSpec((r, c), lambda i: (0, 0))
    return pl.pallas_call(
        _dec_body,
        grid=(_GRID,),
        in_specs=[pl.BlockSpec(memory_space=pltpu.SMEM),
                  blk(_BM, 15),
                  cst(_N, 15),
                  cst(_NATOM, 128), cst(1, 128), cst(128, 128),
                  cst(1, 128)],
        out_specs=[blk(_BM, 128),
                   pl.BlockSpec((139,), lambda i: (0,))],
        out_shape=[jax.ShapeDtypeStruct((_ND, 128), _f32),
                   jax.ShapeDtypeStruct((139,), _f32)],
        scratch_shapes=[pltpu.VMEM((1, 128), _f32),
                        pltpu.VMEM((1, _NATOM), _f32)],
    )(bfs, x_l, x_l, Wd1, bd1[None, :], Wd2, bd2[None, :])


def _lig_body(q_ref, y2_ref, invl_ref, xl_ref,
              wl2_ref, bl2_ref, wf_ref, bf_ref,
              lp_ref, hi_ref, zlsum_ref, labA_ref):
    i = pl.program_id(0)
    rowid = lax.broadcasted_iota(jnp.int32, (_BM, 1), 0) + i * _BM

    @pl.when(i == 0)
    def _():
        lp_ref[...] = jnp.zeros_like(lp_ref)
        zlsum_ref[...] = jnp.zeros_like(zlsum_ref)
        labA_ref[...] = jnp.zeros_like(labA_ref)

    lab = jnp.where(rowid < _N, xl_ref[...][:, 4:], 0.0)
    invl = invl_ref[...]
    aggl2 = invl * jnp.concatenate(
        [q_ref[0] + y2_ref[0], q_ref[1] + y2_ref[1]], axis=1)
    zl = jnp.dot(aggl2, wl2_ref[...], preferred_element_type=_f32) \
        + bl2_ref[...]
    lmask = jnp.where(
        lax.broadcasted_iota(jnp.int32, (1, _NATOM), 1) == _NATOM - 1,
        -1e9, 0.0)
    logits = jnp.dot(zl, wf_ref[...], preferred_element_type=_f32) \
        + bf_ref[...] + lmask
    m = jnp.max(logits, axis=1, keepdims=True)
    e = jnp.exp(logits - m)
    num = jnp.sum(e * lab, axis=1, keepdims=True)
    den = jnp.sum(e, axis=1, keepdims=True)
    lig_mask = rowid < _N
    inner = jnp.where(lig_mask, num / den, 1.0)
    lp_ref[...] += jnp.sum(jnp.log(inner), axis=0, keepdims=True)
    zlsum_ref[...] += jnp.sum(jnp.where(lig_mask, zl, 0.0), axis=0,
                              keepdims=True)
    labA_ref[...] += jnp.sum(jnp.where(lig_mask, lab, 0.0), axis=0,
                             keepdims=True)

    @pl.when(i == _GRID - 1)
    def _():
        hi = jnp.concatenate([zlsum_ref[...] / _N, labA_ref[...] / _N],
                             axis=1)
        hi_ref[...] = hi[0]


def _lig_call(aggl2, y23, invl, x_l, Wl2, bl2, Wf, bf):
    blk = lambda r, c: pl.BlockSpec((r, c), lambda i: (i, 0))
    blk3 = lambda c: pl.BlockSpec((_NC, _BM, c), lambda i: (0, i, 0))
    cst = lambda r, c: pl.BlockSpec((r, c), lambda i: (0, 0))
    return pl.pallas_call(
        _lig_body,
        grid=(_GRID,),
        in_specs=[blk3(_HD), blk3(_HD), blk(_BM, 1), blk(_BM, 15),
                  cst(128, 128), cst(1, 128), cst(128, _NATOM),
                  cst(1, _NATOM)],
        out_specs=[cst(1, 1),
                   pl.BlockSpec((139,), lambda i: (0,))],
        out_shape=[jax.ShapeDtypeStruct((1, 1), _f32),
                   jax.ShapeDtypeStruct((139,), _f32)],
        scratch_shapes=[pltpu.VMEM((1, 128), _f32),
                        pltpu.VMEM((1, _NATOM), _f32)],
    )(aggl2, y23, invl, x_l, Wl2, bl2[None, :], Wf, bf[None, :])


def kernel(x_p, edge_index_p, x_l, edge_index_l, bfs_init, Wp1, bp1, Wp2,
           bp2, Wl1, bl1, Wl2, bl2, Wd1, bd1, Wd2, bd2, Wf, bf):
    ep4 = edge_index_p.reshape(2, _NS, _NCHP, _CH)
    el4 = edge_index_l.reshape(2, _NS, _NCHP, _CH)

    z_v, H_t = _dec_call(bfs_init, x_l, Wd1, bd1, Wd2, bd2)

    dp, dl = _deg_call(ep4, el4)
    degp = (dp[0] + dp[1] + 1.0)[:, None]
    degl = (dl[0] + dl[1] + 1.0)[:, None]

    yp3, yl, invp, invl = _prep_call(degp, degl, x_p, x_l)

    aggp, gmat, aggl = _mid_call(yp3, ep4, invp.reshape(_NPAD), yl, el4)
    gcol = (gmat[0] + gmat[1])[:, None]

    y23, z_pocket = _mid_tc_call(aggp, yp3, invp, gcol, aggl, yl, invl,
                                 Wp1, bp1, Wp2, bp2, Wl1, bl1)

    (aggl2,) = _l2_call(y23, el4)

    lp, H_init = _lig_call(aggl2, y23, invl, x_l, Wl2, bl2, Wf, bf)

    return (lp[0, 0], z_pocket, z_v, H_init, H_t)
